# baseline (speedup 1.0000x reference)
import functools

import jax
import jax.numpy as jnp
from jax import lax
from jax.experimental import pallas as pl
from jax.experimental.pallas import tpu as pltpu
from jax.experimental.pallas import tpu_sc as plsc

N_NODES = 50000
NUM_EDGE = 1600000
NUM_SAMPLE = 800000

_ROWS = NUM_EDGE // 128

_NT = 16
_CHUNK = NUM_EDGE // _NT
_W = 10000
_NWIN = _CHUNK // _W
_NVREG = _W // 16
_NB = 256
_NPASS = 4


def _score_body(m_ref, prob_ref, gum_ref, s_ref, pn_ref, key_ref):
    m = m_ref[0, 0]
    pn = prob_ref[...] / m
    pn_ref[...] = pn
    s = jnp.log(pn) + gum_ref[...]
    s_ref[...] = s
    b = pltpu.bitcast(s, jnp.int32)
    key_ref[...] = jnp.where(b >= 0, jnp.int32(0x7FFFFFFF) - b, b)


def _scores(prob, m, gumbel):
    prob2 = prob.reshape(_ROWS, 128)
    gum2 = gumbel.reshape(_ROWS, 128)
    m1 = m.reshape(1, 1)
    s, pn, key = pl.pallas_call(
        _score_body,
        out_shape=(
            jax.ShapeDtypeStruct((_ROWS, 128), jnp.float32),
            jax.ShapeDtypeStruct((_ROWS, 128), jnp.float32),
            jax.ShapeDtypeStruct((_ROWS, 128), jnp.int32),
        ),
        in_specs=[
            pl.BlockSpec(memory_space=pltpu.SMEM),
            pl.BlockSpec(memory_space=pltpu.VMEM),
            pl.BlockSpec(memory_space=pltpu.VMEM),
        ],
        out_specs=(
            pl.BlockSpec(memory_space=pltpu.VMEM),
            pl.BlockSpec(memory_space=pltpu.VMEM),
            pl.BlockSpec(memory_space=pltpu.VMEM),
        ),
    )(m1, prob2, gum2)
    return s.reshape(NUM_EDGE), pn.reshape(NUM_EDGE), key.reshape(NUM_EDGE)


def _one_pass(shift, write_keys, kin, pin, kout, pout, tid,
              kbuf, pbuf, kob, pob, posb, hist2, cur, hmatbuf, scr, hmat):
    lane = lax.iota(jnp.int32, 16)
    base = tid * jnp.int32(_CHUNK)
    shift = jnp.int32(shift)

    zero16 = jnp.zeros((16,), jnp.int32)
    ones16 = jnp.ones((16,), jnp.int32)

    def zero_vreg(j, off):
        hist2[pl.ds(off, 16)] = zero16
        return off + 16

    lax.fori_loop(0, _NB, zero_vreg, jnp.int32(0))

    def hist_vreg(j, off):
        k = kbuf[pl.ds(off, 16)]
        dig = lax.shift_right_logical(k, shift) & 255
        plsc.addupdate_scatter(hist2, [lane * _NB + dig], ones16)
        return off + 16

    def hist_win(w, off):
        pltpu.sync_copy(kin.at[pl.ds(pl.multiple_of(base + off, 8), _W)], kbuf)
        lax.fori_loop(0, _NVREG, hist_vreg, jnp.int32(0))
        return off + _W

    lax.fori_loop(0, _NWIN, hist_win, jnp.int32(0))

    def red_lane(l, c):
        loff, g16, tot = c
        return (loff + _NB, g16, tot + hist2[pl.ds(loff + g16, 16)])

    def red_grp(g, g16):
        _, _, tot = lax.fori_loop(0, 16, red_lane, (jnp.int32(0), g16, zero16))
        kob[pl.ds(g16, 16)] = tot
        return g16 + 16

    lax.fori_loop(0, _NB // 16, red_grp, jnp.int32(0))

    pltpu.sync_copy(kob.at[pl.ds(0, _NB)], hmat.at[tid])
    plsc.subcore_barrier()
    pltpu.sync_copy(hmat, hmatbuf)
    plsc.subcore_barrier()

    def scan_tile(t, c):
        ti, g16, tot, pre = c
        row = hmatbuf[ti, pl.ds(g16, 16)]
        return (ti + 1, g16, tot + row,
                pre + jnp.where(ti < tid, row, zero16))

    def scan_grp(g, c):
        g16, carry = c
        _, _, tot, pre = lax.fori_loop(
            0, _NT, scan_tile, (jnp.int32(0), g16, zero16, zero16))
        cs = plsc.cumsum(tot)
        excl = (cs - tot) + carry
        cur[pl.ds(g16, 16)] = excl + pre
        return (g16 + 16, carry + lax.reduce_max(cs, (0,)))

    lax.fori_loop(0, _NB // 16, scan_grp, (jnp.int32(0), jnp.int32(0)))

    big = jnp.full((16,), 16, jnp.int32)
    lane_m1 = jnp.maximum(lane - 1, 0)
    lane_p1 = jnp.minimum(lane + 1, 15)

    def rank_vreg(j, off):
        k = kbuf[pl.ds(off, 16)]
        p = pbuf[pl.ds(off, 16)]
        dig = lax.shift_right_logical(k, shift) & 255
        ck = dig * 16 + lane
        _, k_s = plsc.sort_key_val(ck, k)
        ck_s, p_s = plsc.sort_key_val(ck, p)
        dig_s = lax.shift_right_logical(ck_s, jnp.int32(4))
        scr[...] = dig_s
        prev = plsc.load_gather(scr, [lane_m1])
        head = (lane == 0) | (dig_s != prev)
        start = plsc.cummax(jnp.where(head, lane, 0))
        rank = lane - start
        hidx = jnp.where(head, lane, big)
        scr[...] = hidx
        hshift = plsc.load_gather(scr, [lane_p1])
        hshift = jnp.where(lane == 15, big, hshift)
        sufmin = -lax.rev(plsc.cummax(lax.rev(-hshift, (0,))), (0,))
        cnt = sufmin - lane
        c = plsc.load_gather(cur, [dig_s])
        pos = c + rank
        if write_keys:
            kob[pl.ds(off, 16)] = k_s
        pob[pl.ds(off, 16)] = p_s
        posb[pl.ds(off, 16)] = pos
        plsc.addupdate_scatter(cur, [dig_s], cnt, mask=head)
        return off + 16

    def rank_win(w, off):
        pltpu.sync_copy(kin.at[pl.ds(pl.multiple_of(base + off, 8), _W)], kbuf)
        pltpu.sync_copy(pin.at[pl.ds(pl.multiple_of(base + off, 8), _W)], pbuf)
        lax.fori_loop(0, _NVREG, rank_vreg, jnp.int32(0))
        if write_keys:
            pltpu.sync_copy(kob, kout.at[posb])
        pltpu.sync_copy(pob, pout.at[posb])
        return off + _W

    lax.fori_loop(0, _NWIN, rank_win, jnp.int32(0))

    plsc.subcore_barrier()


def _sort_body(kin, pin, pout_hbm, kscr, pscr, kscr2,
               kbuf, pbuf, kob, pob, posb, hist2, cur, hmatbuf, scr, hmat):
    cid = lax.axis_index("c")
    tid = lax.axis_index("s")

    @pl.when(cid == 0)
    def _():
        args = (tid, kbuf, pbuf, kob, pob, posb, hist2, cur, hmatbuf, scr,
                hmat)
        _one_pass(0, True, kin, pin, kscr, pscr, *args)
        _one_pass(8, True, kscr, pscr, kscr2, pout_hbm, *args)
        _one_pass(16, True, kscr2, pout_hbm, kscr, pscr, *args)
        _one_pass(24, False, kscr, pscr, kscr2, pout_hbm, *args)


def _sc_radix_sort(keys, payload):
    mesh = plsc.VectorSubcoreMesh(core_axis_name="c", subcore_axis_name="s")
    f = pl.kernel(
        _sort_body,
        out_type=jax.ShapeDtypeStruct((NUM_EDGE,), jnp.int32),
        mesh=mesh,
        compiler_params=pltpu.CompilerParams(needs_layout_passes=False),
        scratch_types=[
            pltpu.HBM((NUM_EDGE,), jnp.int32),
            pltpu.HBM((NUM_EDGE,), jnp.int32),
            pltpu.HBM((NUM_EDGE,), jnp.int32),
            pltpu.VMEM((_W,), jnp.int32),
            pltpu.VMEM((_W,), jnp.int32),
            pltpu.VMEM((_W,), jnp.int32),
            pltpu.VMEM((_W,), jnp.int32),
            pltpu.VMEM((_W,), jnp.int32),
            pltpu.VMEM((_NB * 16,), jnp.int32),
            pltpu.VMEM((_NB,), jnp.int32),
            pltpu.VMEM((_NT, _NB), jnp.int32),
            pltpu.VMEM((16,), jnp.int32),
            pltpu.VMEM_SHARED((_NT, _NB), jnp.int32),
        ],
    )
    return f(keys, payload)


def kernel(edge_index, edge_weight):
    node_in = edge_index[0].astype(jnp.int32)
    node_out = edge_index[1].astype(jnp.int32)

    degree_in = jax.ops.segment_sum(edge_weight, node_in, num_segments=N_NODES)
    degree_out = jax.ops.segment_sum(edge_weight, node_out, num_segments=N_NODES)

    prob = 1.0 / jnp.take(degree_out, node_out) + 1.0 / jnp.take(degree_in, node_in)
    m = jnp.mean(prob)

    u = jax.random.uniform(jax.random.key(42), (NUM_EDGE,), dtype=jnp.float32,
                           minval=1e-20, maxval=1.0)
    gumbel = -jnp.log(-jnp.log(u))

    s, prob_n, key = _scores(prob, m, gumbel)

    perm = _sc_radix_sort(key, jnp.arange(NUM_EDGE, dtype=jnp.int32))
    index = perm[:NUM_SAMPLE]

    new_edge_index = jnp.take(edge_index, index, axis=1)
    new_edge_weight = jnp.take(edge_weight, index) / (
        NUM_SAMPLE * jnp.take(prob_n, index) / NUM_EDGE)
    return new_edge_index, new_edge_weight


# speedup vs baseline: 1.0002x; 1.0002x over previous
import functools

import jax
import jax.numpy as jnp
from jax import lax
from jax.experimental import pallas as pl
from jax.experimental.pallas import tpu as pltpu
from jax.experimental.pallas import tpu_sc as plsc

N_NODES = 50000
NUM_EDGE = 1600000
NUM_SAMPLE = 800000

_ROWS = NUM_EDGE // 128

_NT = 16
_CHUNK = NUM_EDGE // _NT
_W = 10000
_NWIN = _CHUNK // _W
_NVREG = _W // 16
_NB = 256
_NPASS = 4


def _score_body(m_ref, prob_ref, gum_ref, s_ref, pn_ref, key_ref):
    m = m_ref[0, 0]
    pn = prob_ref[...] / m
    pn_ref[...] = pn
    s = jnp.log(pn) + gum_ref[...]
    s_ref[...] = s
    b = pltpu.bitcast(s, jnp.int32)
    key_ref[...] = jnp.where(b >= 0, jnp.int32(0x7FFFFFFF) - b, b)


def _scores(prob, m, gumbel):
    prob2 = prob.reshape(_ROWS, 128)
    gum2 = gumbel.reshape(_ROWS, 128)
    m1 = m.reshape(1, 1)
    s, pn, key = pl.pallas_call(
        _score_body,
        out_shape=(
            jax.ShapeDtypeStruct((_ROWS, 128), jnp.float32),
            jax.ShapeDtypeStruct((_ROWS, 128), jnp.float32),
            jax.ShapeDtypeStruct((_ROWS, 128), jnp.int32),
        ),
        in_specs=[
            pl.BlockSpec(memory_space=pltpu.SMEM),
            pl.BlockSpec(memory_space=pltpu.VMEM),
            pl.BlockSpec(memory_space=pltpu.VMEM),
        ],
        out_specs=(
            pl.BlockSpec(memory_space=pltpu.VMEM),
            pl.BlockSpec(memory_space=pltpu.VMEM),
            pl.BlockSpec(memory_space=pltpu.VMEM),
        ),
    )(m1, prob2, gum2)
    return s.reshape(NUM_EDGE), pn.reshape(NUM_EDGE), key.reshape(NUM_EDGE)


def _one_pass(shift, write_keys, kin, pin, kout, pout, tid,
              kbuf, pbuf, kob, pob, posb, hist2, cur, hmatbuf, scr, hmat):
    lane = lax.iota(jnp.int32, 16)
    base = tid * jnp.int32(_CHUNK)
    shift = jnp.int32(shift)

    zero16 = jnp.zeros((16,), jnp.int32)
    ones16 = jnp.ones((16,), jnp.int32)

    def zero_vreg(j, off):
        hist2[pl.ds(off, 16)] = zero16
        return off + 16

    lax.fori_loop(0, _NB, zero_vreg, jnp.int32(0))

    def hist_vreg(j, off):
        k = kbuf[pl.ds(off, 16)]
        dig = lax.shift_right_logical(k, shift) & 255
        plsc.addupdate_scatter(hist2, [lane * _NB + dig], ones16)
        return off + 16

    def hist_win(w, off):
        pltpu.sync_copy(kin.at[pl.ds(pl.multiple_of(base + off, 8), _W)], kbuf)
        lax.fori_loop(0, _NVREG, hist_vreg, jnp.int32(0))
        return off + _W

    lax.fori_loop(0, _NWIN, hist_win, jnp.int32(0))

    def red_lane(l, c):
        loff, g16, tot = c
        return (loff + _NB, g16, tot + hist2[pl.ds(loff + g16, 16)])

    def red_grp(g, g16):
        _, _, tot = lax.fori_loop(0, 16, red_lane, (jnp.int32(0), g16, zero16))
        kob[pl.ds(g16, 16)] = tot
        return g16 + 16

    lax.fori_loop(0, _NB // 16, red_grp, jnp.int32(0))

    pltpu.sync_copy(kob.at[pl.ds(0, _NB)], hmat.at[tid])
    plsc.subcore_barrier()
    pltpu.sync_copy(hmat, hmatbuf)
    plsc.subcore_barrier()

    def scan_tile(t, c):
        ti, g16, tot, pre = c
        row = hmatbuf[ti, pl.ds(g16, 16)]
        return (ti + 1, g16, tot + row,
                pre + jnp.where(ti < tid, row, zero16))

    def scan_grp(g, c):
        g16, carry = c
        _, _, tot, pre = lax.fori_loop(
            0, _NT, scan_tile, (jnp.int32(0), g16, zero16, zero16))
        cs = plsc.cumsum(tot)
        excl = (cs - tot) + carry
        cur[pl.ds(g16, 16)] = excl + pre
        return (g16 + 16, carry + lax.reduce_max(cs, (0,)))

    lax.fori_loop(0, _NB // 16, scan_grp, (jnp.int32(0), jnp.int32(0)))

    big = jnp.full((16,), 16, jnp.int32)
    lane_m1 = jnp.maximum(lane - 1, 0)
    lane_p1 = jnp.minimum(lane + 1, 15)

    def rank_vreg(j, off):
        k = kbuf[pl.ds(off, 16)]
        p = pbuf[pl.ds(off, 16)]
        dig = lax.shift_right_logical(k, shift) & 255
        ck = dig * 16 + lane
        _, k_s = plsc.sort_key_val(ck, k)
        ck_s, p_s = plsc.sort_key_val(ck, p)
        dig_s = lax.shift_right_logical(ck_s, jnp.int32(4))
        scr[...] = dig_s
        prev = plsc.load_gather(scr, [lane_m1])
        head = (lane == 0) | (dig_s != prev)
        start = plsc.cummax(jnp.where(head, lane, 0))
        rank = lane - start
        hidx = jnp.where(head, lane, big)
        scr[...] = hidx
        hshift = plsc.load_gather(scr, [lane_p1])
        hshift = jnp.where(lane == 15, big, hshift)
        sufmin = -lax.rev(plsc.cummax(lax.rev(-hshift, (0,))), (0,))
        cnt = sufmin - lane
        c = plsc.load_gather(cur, [dig_s])
        pos = c + rank
        if write_keys:
            kob[pl.ds(off, 16)] = k_s
        pob[pl.ds(off, 16)] = p_s
        posb[pl.ds(off, 16)] = pos
        plsc.addupdate_scatter(cur, [dig_s], cnt, mask=head)
        return off + 16

    def rank_win(w, off):
        pltpu.sync_copy(kin.at[pl.ds(pl.multiple_of(base + off, 8), _W)], kbuf)
        pltpu.sync_copy(pin.at[pl.ds(pl.multiple_of(base + off, 8), _W)], pbuf)
        lax.fori_loop(0, _NVREG, rank_vreg, jnp.int32(0))
        if write_keys:
            pltpu.sync_copy(kob, kout.at[posb])
        pltpu.sync_copy(pob, pout.at[posb])
        return off + _W

    lax.fori_loop(0, _NWIN, rank_win, jnp.int32(0))

    plsc.subcore_barrier()


def _sort_body(kin, pin, pout_hbm, kscr, pscr, kscr2,
               kbuf, pbuf, kob, pob, posb, hist2, cur, hmatbuf, scr, hmat):
    cid = lax.axis_index("c")
    tid = lax.axis_index("s")

    @pl.when(cid == 0)
    def _():
        args = (tid, kbuf, pbuf, kob, pob, posb, hist2, cur, hmatbuf, scr,
                hmat)
        _one_pass(0, True, kin, pin, kscr, pscr, *args)
        _one_pass(8, True, kscr, pscr, kscr2, pout_hbm, *args)
        _one_pass(16, True, kscr2, pout_hbm, kscr, pscr, *args)
        _one_pass(24, False, kscr, pscr, kscr2, pout_hbm, *args)


def _sc_radix_sort(keys, payload):
    mesh = plsc.VectorSubcoreMesh(core_axis_name="c", subcore_axis_name="s")
    f = pl.kernel(
        _sort_body,
        out_type=jax.ShapeDtypeStruct((NUM_EDGE,), jnp.int32),
        mesh=mesh,
        compiler_params=pltpu.CompilerParams(needs_layout_passes=False),
        scratch_types=[
            pltpu.HBM((NUM_EDGE,), jnp.int32),
            pltpu.HBM((NUM_EDGE,), jnp.int32),
            pltpu.HBM((NUM_EDGE,), jnp.int32),
            pltpu.VMEM((_W,), jnp.int32),
            pltpu.VMEM((_W,), jnp.int32),
            pltpu.VMEM((_W,), jnp.int32),
            pltpu.VMEM((_W,), jnp.int32),
            pltpu.VMEM((_W,), jnp.int32),
            pltpu.VMEM((_NB * 16,), jnp.int32),
            pltpu.VMEM((_NB,), jnp.int32),
            pltpu.VMEM((_NT, _NB), jnp.int32),
            pltpu.VMEM((16,), jnp.int32),
            pltpu.VMEM_SHARED((_NT, _NB), jnp.int32),
        ],
    )
    return f(keys, payload)


def kernel(edge_index, edge_weight):
    node_in = edge_index[0].astype(jnp.int32)
    node_out = edge_index[1].astype(jnp.int32)

    degree_in = jax.ops.segment_sum(edge_weight, node_in, num_segments=N_NODES)
    degree_out = jax.ops.segment_sum(edge_weight, node_out, num_segments=N_NODES)

    prob = 1.0 / jnp.take(degree_out, node_out) + 1.0 / jnp.take(degree_in, node_in)
    m = jnp.mean(prob)

    u = jax.random.uniform(jax.random.key(42), (NUM_EDGE,), dtype=jnp.float32,
                           minval=1e-20, maxval=1.0)
    gumbel = -jnp.log(-jnp.log(u))

    s, prob_n, key = _scores(prob, m, gumbel)

    perm = _sc_radix_sort(key, jnp.arange(NUM_EDGE, dtype=jnp.int32))
    index = perm[:NUM_SAMPLE]

    new_in = jnp.take(node_in, index)
    new_out = jnp.take(node_out, index)
    new_edge_index = jnp.stack([new_in, new_out]).astype(jnp.int64)
    new_edge_weight = jnp.take(edge_weight, index) / (
        NUM_SAMPLE * jnp.take(prob_n, index) / NUM_EDGE)
    return new_edge_index, new_edge_weight


# speedup vs baseline: 2.1192x; 2.1188x over previous
import functools

import jax
import jax.numpy as jnp
from jax import lax
from jax.experimental import pallas as pl
from jax.experimental.pallas import tpu as pltpu
from jax.experimental.pallas import tpu_sc as plsc

N_NODES = 50000
NUM_EDGE = 1600000
NUM_SAMPLE = 800000

_ROWS = NUM_EDGE // 128

_NT = 16
_CHUNK = NUM_EDGE // _NT
_W = 10000
_NWIN = _CHUNK // _W
_NVREG = _W // 16
_NB = 256
_NPASS = 4


def _score_body(m_ref, prob_ref, gum_ref, s_ref, pn_ref, key_ref):
    m = m_ref[0, 0]
    pn = prob_ref[...] / m
    pn_ref[...] = pn
    s = jnp.log(pn) + gum_ref[...]
    s_ref[...] = s
    b = pltpu.bitcast(s, jnp.int32)
    key_ref[...] = jnp.where(b >= 0, jnp.int32(0x7FFFFFFF) - b, b)


def _scores(prob, m, gumbel):
    prob2 = prob.reshape(_ROWS, 128)
    gum2 = gumbel.reshape(_ROWS, 128)
    m1 = m.reshape(1, 1)
    s, pn, key = pl.pallas_call(
        _score_body,
        out_shape=(
            jax.ShapeDtypeStruct((_ROWS, 128), jnp.float32),
            jax.ShapeDtypeStruct((_ROWS, 128), jnp.float32),
            jax.ShapeDtypeStruct((_ROWS, 128), jnp.int32),
        ),
        in_specs=[
            pl.BlockSpec(memory_space=pltpu.SMEM),
            pl.BlockSpec(memory_space=pltpu.VMEM),
            pl.BlockSpec(memory_space=pltpu.VMEM),
        ],
        out_specs=(
            pl.BlockSpec(memory_space=pltpu.VMEM),
            pl.BlockSpec(memory_space=pltpu.VMEM),
            pl.BlockSpec(memory_space=pltpu.VMEM),
        ),
    )(m1, prob2, gum2)
    return s.reshape(NUM_EDGE), pn.reshape(NUM_EDGE), key.reshape(NUM_EDGE)


_GW = 10000
_GCHUNK = NUM_EDGE // 32
_GNWIN = _GCHUNK // _GW


def _prob_body(rin, rout, nin, nout, prob_hbm,
               nbuf, mbuf, gbuf, hbuf, obuf, sem):
    cid = lax.axis_index("c")
    sid = lax.axis_index("s")
    wid = sid * 2 + cid
    base = wid * jnp.int32(_GCHUNK)

    def add_vreg(j, off):
        obuf[pl.ds(off, 16)] = gbuf[pl.ds(off, 16)] + hbuf[pl.ds(off, 16)]
        return off + 16

    def win(w, off):
        start = pl.multiple_of(base + off, 8)
        pltpu.sync_copy(nin.at[pl.ds(start, _GW)], nbuf)
        pltpu.sync_copy(nout.at[pl.ds(start, _GW)], mbuf)
        cin = pltpu.async_copy(rin.at[nbuf], gbuf, sem)
        cin.wait()
        cout = pltpu.async_copy(rout.at[mbuf], hbuf, sem)
        cout.wait()
        lax.fori_loop(0, _GW // 16, add_vreg, jnp.int32(0))
        pltpu.sync_copy(obuf, prob_hbm.at[pl.ds(start, _GW)])
        return off + _GW

    lax.fori_loop(0, _GNWIN, win, jnp.int32(0))


def _sc_prob(r_in, r_out, node_in, node_out):
    mesh = plsc.VectorSubcoreMesh(core_axis_name="c", subcore_axis_name="s")
    f = pl.kernel(
        _prob_body,
        out_type=jax.ShapeDtypeStruct((NUM_EDGE,), jnp.float32),
        mesh=mesh,
        compiler_params=pltpu.CompilerParams(needs_layout_passes=False),
        scratch_types=[
            pltpu.VMEM((_GW,), jnp.int32),
            pltpu.VMEM((_GW,), jnp.int32),
            pltpu.VMEM((_GW,), jnp.float32),
            pltpu.VMEM((_GW,), jnp.float32),
            pltpu.VMEM((_GW,), jnp.float32),
            pltpu.SemaphoreType.DMA,
        ],
    )
    return f(r_in, r_out, node_in, node_out)


def _one_pass(shift, write_keys, kin, pin, kout, pout, tid,
              kbuf, pbuf, kob, pob, posb, hist2, cur, hmatbuf, scr, hmat):
    lane = lax.iota(jnp.int32, 16)
    base = tid * jnp.int32(_CHUNK)
    shift = jnp.int32(shift)

    zero16 = jnp.zeros((16,), jnp.int32)
    ones16 = jnp.ones((16,), jnp.int32)

    def zero_vreg(j, off):
        hist2[pl.ds(off, 16)] = zero16
        return off + 16

    lax.fori_loop(0, _NB, zero_vreg, jnp.int32(0))

    def hist_vreg(j, off):
        k = kbuf[pl.ds(off, 16)]
        dig = lax.shift_right_logical(k, shift) & 255
        plsc.addupdate_scatter(hist2, [lane * _NB + dig], ones16)
        return off + 16

    def hist_win(w, off):
        pltpu.sync_copy(kin.at[pl.ds(pl.multiple_of(base + off, 8), _W)], kbuf)
        lax.fori_loop(0, _NVREG, hist_vreg, jnp.int32(0))
        return off + _W

    lax.fori_loop(0, _NWIN, hist_win, jnp.int32(0))

    def red_lane(l, c):
        loff, g16, tot = c
        return (loff + _NB, g16, tot + hist2[pl.ds(loff + g16, 16)])

    def red_grp(g, g16):
        _, _, tot = lax.fori_loop(0, 16, red_lane, (jnp.int32(0), g16, zero16))
        kob[pl.ds(g16, 16)] = tot
        return g16 + 16

    lax.fori_loop(0, _NB // 16, red_grp, jnp.int32(0))

    pltpu.sync_copy(kob.at[pl.ds(0, _NB)], hmat.at[tid])
    plsc.subcore_barrier()
    pltpu.sync_copy(hmat, hmatbuf)
    plsc.subcore_barrier()

    def scan_tile(t, c):
        ti, g16, tot, pre = c
        row = hmatbuf[ti, pl.ds(g16, 16)]
        return (ti + 1, g16, tot + row,
                pre + jnp.where(ti < tid, row, zero16))

    def scan_grp(g, c):
        g16, carry = c
        _, _, tot, pre = lax.fori_loop(
            0, _NT, scan_tile, (jnp.int32(0), g16, zero16, zero16))
        cs = plsc.cumsum(tot)
        excl = (cs - tot) + carry
        cur[pl.ds(g16, 16)] = excl + pre
        return (g16 + 16, carry + lax.reduce_max(cs, (0,)))

    lax.fori_loop(0, _NB // 16, scan_grp, (jnp.int32(0), jnp.int32(0)))

    big = jnp.full((16,), 16, jnp.int32)
    lane_m1 = jnp.maximum(lane - 1, 0)
    lane_p1 = jnp.minimum(lane + 1, 15)

    def rank_vreg(j, off):
        k = kbuf[pl.ds(off, 16)]
        p = pbuf[pl.ds(off, 16)]
        dig = lax.shift_right_logical(k, shift) & 255
        ck = dig * 16 + lane
        _, k_s = plsc.sort_key_val(ck, k)
        ck_s, p_s = plsc.sort_key_val(ck, p)
        dig_s = lax.shift_right_logical(ck_s, jnp.int32(4))
        scr[...] = dig_s
        prev = plsc.load_gather(scr, [lane_m1])
        head = (lane == 0) | (dig_s != prev)
        start = plsc.cummax(jnp.where(head, lane, 0))
        rank = lane - start
        hidx = jnp.where(head, lane, big)
        scr[...] = hidx
        hshift = plsc.load_gather(scr, [lane_p1])
        hshift = jnp.where(lane == 15, big, hshift)
        sufmin = -lax.rev(plsc.cummax(lax.rev(-hshift, (0,))), (0,))
        cnt = sufmin - lane
        c = plsc.load_gather(cur, [dig_s])
        pos = c + rank
        if write_keys:
            kob[pl.ds(off, 16)] = k_s
        pob[pl.ds(off, 16)] = p_s
        posb[pl.ds(off, 16)] = pos
        plsc.addupdate_scatter(cur, [dig_s], cnt, mask=head)
        return off + 16

    def rank_win(w, off):
        pltpu.sync_copy(kin.at[pl.ds(pl.multiple_of(base + off, 8), _W)], kbuf)
        pltpu.sync_copy(pin.at[pl.ds(pl.multiple_of(base + off, 8), _W)], pbuf)
        lax.fori_loop(0, _NVREG, rank_vreg, jnp.int32(0))
        if write_keys:
            pltpu.sync_copy(kob, kout.at[posb])
        pltpu.sync_copy(pob, pout.at[posb])
        return off + _W

    lax.fori_loop(0, _NWIN, rank_win, jnp.int32(0))

    plsc.subcore_barrier()


def _sort_body(kin, pin, pout_hbm, kscr, pscr, kscr2,
               kbuf, pbuf, kob, pob, posb, hist2, cur, hmatbuf, scr, hmat):
    cid = lax.axis_index("c")
    tid = lax.axis_index("s")

    @pl.when(cid == 0)
    def _():
        args = (tid, kbuf, pbuf, kob, pob, posb, hist2, cur, hmatbuf, scr,
                hmat)
        _one_pass(0, True, kin, pin, kscr, pscr, *args)
        _one_pass(8, True, kscr, pscr, kscr2, pout_hbm, *args)
        _one_pass(16, True, kscr2, pout_hbm, kscr, pscr, *args)
        _one_pass(24, False, kscr, pscr, kscr2, pout_hbm, *args)


def _sc_radix_sort(keys, payload):
    mesh = plsc.VectorSubcoreMesh(core_axis_name="c", subcore_axis_name="s")
    f = pl.kernel(
        _sort_body,
        out_type=jax.ShapeDtypeStruct((NUM_EDGE,), jnp.int32),
        mesh=mesh,
        compiler_params=pltpu.CompilerParams(needs_layout_passes=False),
        scratch_types=[
            pltpu.HBM((NUM_EDGE,), jnp.int32),
            pltpu.HBM((NUM_EDGE,), jnp.int32),
            pltpu.HBM((NUM_EDGE,), jnp.int32),
            pltpu.VMEM((_W,), jnp.int32),
            pltpu.VMEM((_W,), jnp.int32),
            pltpu.VMEM((_W,), jnp.int32),
            pltpu.VMEM((_W,), jnp.int32),
            pltpu.VMEM((_W,), jnp.int32),
            pltpu.VMEM((_NB * 16,), jnp.int32),
            pltpu.VMEM((_NB,), jnp.int32),
            pltpu.VMEM((_NT, _NB), jnp.int32),
            pltpu.VMEM((16,), jnp.int32),
            pltpu.VMEM_SHARED((_NT, _NB), jnp.int32),
        ],
    )
    return f(keys, payload)


def kernel(edge_index, edge_weight):
    node_in = edge_index[0].astype(jnp.int32)
    node_out = edge_index[1].astype(jnp.int32)

    degree_in = jax.ops.segment_sum(edge_weight, node_in, num_segments=N_NODES)
    degree_out = jax.ops.segment_sum(edge_weight, node_out, num_segments=N_NODES)

    prob = _sc_prob(1.0 / degree_in, 1.0 / degree_out, node_in, node_out)
    m = jnp.mean(prob)

    u = jax.random.uniform(jax.random.key(42), (NUM_EDGE,), dtype=jnp.float32,
                           minval=1e-20, maxval=1.0)
    gumbel = -jnp.log(-jnp.log(u))

    s, prob_n, key = _scores(prob, m, gumbel)

    perm = _sc_radix_sort(key, jnp.arange(NUM_EDGE, dtype=jnp.int32))
    index = perm[:NUM_SAMPLE]

    new_in = jnp.take(node_in, index)
    new_out = jnp.take(node_out, index)
    new_edge_index = jnp.stack([new_in, new_out]).astype(jnp.int64)
    new_edge_weight = jnp.take(edge_weight, index) / (
        NUM_SAMPLE * jnp.take(prob_n, index) / NUM_EDGE)
    return new_edge_index, new_edge_weight


# speedup vs baseline: 2.1243x; 1.0024x over previous
import functools

import jax
import jax.numpy as jnp
from jax import lax
from jax.experimental import pallas as pl
from jax.experimental.pallas import tpu as pltpu
from jax.experimental.pallas import tpu_sc as plsc

N_NODES = 50000
NUM_EDGE = 1600000
NUM_SAMPLE = 800000

_ROWS = NUM_EDGE // 128

_NT = 16
_CHUNK = NUM_EDGE // _NT
_W = 10000
_NWIN = _CHUNK // _W
_NVREG = _W // 16
_NB = 256
_NPASS = 4


def _score_body(m_ref, prob_ref, gum_ref, s_ref, pn_ref, key_ref):
    m = m_ref[0, 0]
    pn = prob_ref[...] / m
    pn_ref[...] = pn
    s = jnp.log(pn) + gum_ref[...]
    s_ref[...] = s
    b = pltpu.bitcast(s, jnp.int32)
    key_ref[...] = jnp.where(b >= 0, jnp.int32(0x7FFFFFFF) - b, b)


def _scores(prob, m, gumbel):
    prob2 = prob.reshape(_ROWS, 128)
    gum2 = gumbel.reshape(_ROWS, 128)
    m1 = m.reshape(1, 1)
    s, pn, key = pl.pallas_call(
        _score_body,
        out_shape=(
            jax.ShapeDtypeStruct((_ROWS, 128), jnp.float32),
            jax.ShapeDtypeStruct((_ROWS, 128), jnp.float32),
            jax.ShapeDtypeStruct((_ROWS, 128), jnp.int32),
        ),
        in_specs=[
            pl.BlockSpec(memory_space=pltpu.SMEM),
            pl.BlockSpec(memory_space=pltpu.VMEM),
            pl.BlockSpec(memory_space=pltpu.VMEM),
        ],
        out_specs=(
            pl.BlockSpec(memory_space=pltpu.VMEM),
            pl.BlockSpec(memory_space=pltpu.VMEM),
            pl.BlockSpec(memory_space=pltpu.VMEM),
        ),
    )(m1, prob2, gum2)
    return s.reshape(NUM_EDGE), pn.reshape(NUM_EDGE), key.reshape(NUM_EDGE)


_GW = 10000
_GCHUNK = NUM_EDGE // 32
_GNWIN = _GCHUNK // _GW


def _prob_body(rin, rout, nin, nout, prob_hbm,
               nbuf, mbuf, gbuf, hbuf, obuf, sem):
    cid = lax.axis_index("c")
    sid = lax.axis_index("s")
    wid = sid * 2 + cid
    base = wid * jnp.int32(_GCHUNK)

    def add_vreg(j, off):
        obuf[pl.ds(off, 16)] = gbuf[pl.ds(off, 16)] + hbuf[pl.ds(off, 16)]
        return off + 16

    def win(w, off):
        start = pl.multiple_of(base + off, 8)
        pltpu.sync_copy(nin.at[pl.ds(start, _GW)], nbuf)
        pltpu.sync_copy(nout.at[pl.ds(start, _GW)], mbuf)
        cin = pltpu.async_copy(rin.at[nbuf], gbuf, sem)
        cin.wait()
        cout = pltpu.async_copy(rout.at[mbuf], hbuf, sem)
        cout.wait()
        lax.fori_loop(0, _GW // 16, add_vreg, jnp.int32(0))
        pltpu.sync_copy(obuf, prob_hbm.at[pl.ds(start, _GW)])
        return off + _GW

    lax.fori_loop(0, _GNWIN, win, jnp.int32(0))


def _sc_prob(r_in, r_out, node_in, node_out):
    mesh = plsc.VectorSubcoreMesh(core_axis_name="c", subcore_axis_name="s")
    f = pl.kernel(
        _prob_body,
        out_type=jax.ShapeDtypeStruct((NUM_EDGE,), jnp.float32),
        mesh=mesh,
        compiler_params=pltpu.CompilerParams(needs_layout_passes=False),
        scratch_types=[
            pltpu.VMEM((_GW,), jnp.int32),
            pltpu.VMEM((_GW,), jnp.int32),
            pltpu.VMEM((_GW,), jnp.float32),
            pltpu.VMEM((_GW,), jnp.float32),
            pltpu.VMEM((_GW,), jnp.float32),
            pltpu.SemaphoreType.DMA,
        ],
    )
    return f(r_in, r_out, node_in, node_out)


def _one_pass(shift, write_keys, kin, pin, kout, pout, tid,
              kbuf, pbuf, kob, pob, posb, hist2, cur, hmatbuf, scr, hmat):
    lane = lax.iota(jnp.int32, 16)
    base = tid * jnp.int32(_CHUNK)
    shift = jnp.int32(shift)

    zero16 = jnp.zeros((16,), jnp.int32)
    ones16 = jnp.ones((16,), jnp.int32)

    def zero_vreg(j, off):
        hist2[pl.ds(off, 16)] = zero16
        return off + 16

    lax.fori_loop(0, _NB, zero_vreg, jnp.int32(0))

    def hist_vreg(j, off):
        k = kbuf[pl.ds(off, 16)]
        dig = lax.shift_right_logical(k, shift) & 255
        plsc.addupdate_scatter(hist2, [lane * _NB + dig], ones16)
        return off + 16

    def hist_win(w, off):
        pltpu.sync_copy(kin.at[pl.ds(pl.multiple_of(base + off, 8), _W)], kbuf)
        lax.fori_loop(0, _NVREG, hist_vreg, jnp.int32(0))
        return off + _W

    lax.fori_loop(0, _NWIN, hist_win, jnp.int32(0))

    def red_lane(l, c):
        loff, g16, tot = c
        return (loff + _NB, g16, tot + hist2[pl.ds(loff + g16, 16)])

    def red_grp(g, g16):
        _, _, tot = lax.fori_loop(0, 16, red_lane, (jnp.int32(0), g16, zero16))
        kob[pl.ds(g16, 16)] = tot
        return g16 + 16

    lax.fori_loop(0, _NB // 16, red_grp, jnp.int32(0))

    pltpu.sync_copy(kob.at[pl.ds(0, _NB)], hmat.at[tid])
    plsc.subcore_barrier()
    pltpu.sync_copy(hmat, hmatbuf)
    plsc.subcore_barrier()

    def scan_tile(t, c):
        ti, g16, tot, pre = c
        row = hmatbuf[ti, pl.ds(g16, 16)]
        return (ti + 1, g16, tot + row,
                pre + jnp.where(ti < tid, row, zero16))

    def scan_grp(g, c):
        g16, carry = c
        _, _, tot, pre = lax.fori_loop(
            0, _NT, scan_tile, (jnp.int32(0), g16, zero16, zero16))
        cs = plsc.cumsum(tot)
        excl = (cs - tot) + carry
        cur[pl.ds(g16, 16)] = excl + pre
        return (g16 + 16, carry + lax.reduce_max(cs, (0,)))

    lax.fori_loop(0, _NB // 16, scan_grp, (jnp.int32(0), jnp.int32(0)))

    big = jnp.full((16,), 16, jnp.int32)
    lane_m1 = jnp.maximum(lane - 1, 0)
    lane_p1 = jnp.minimum(lane + 1, 15)

    def rank_vreg(j, off):
        k = kbuf[pl.ds(off, 16)]
        p = pbuf[pl.ds(off, 16)]
        dig = lax.shift_right_logical(k, shift) & 255
        ck = dig * 16 + lane
        _, k_s = plsc.sort_key_val(ck, k)
        ck_s, p_s = plsc.sort_key_val(ck, p)
        dig_s = lax.shift_right_logical(ck_s, jnp.int32(4))
        scr[...] = dig_s
        prev = plsc.load_gather(scr, [lane_m1])
        head = (lane == 0) | (dig_s != prev)
        start = plsc.cummax(jnp.where(head, lane, 0))
        rank = lane - start
        c = plsc.load_gather(cur, [dig_s])
        pos = c + rank
        if write_keys:
            kob[pl.ds(off, 16)] = k_s
        pob[pl.ds(off, 16)] = p_s
        posb[pl.ds(off, 16)] = pos
        plsc.addupdate_scatter(cur, [dig_s], ones16)
        return off + 16

    def rank_win(w, off):
        pltpu.sync_copy(kin.at[pl.ds(pl.multiple_of(base + off, 8), _W)], kbuf)
        pltpu.sync_copy(pin.at[pl.ds(pl.multiple_of(base + off, 8), _W)], pbuf)
        lax.fori_loop(0, _NVREG, rank_vreg, jnp.int32(0))
        if write_keys:
            pltpu.sync_copy(kob, kout.at[posb])
        pltpu.sync_copy(pob, pout.at[posb])
        return off + _W

    lax.fori_loop(0, _NWIN, rank_win, jnp.int32(0))

    plsc.subcore_barrier()


def _sort_body(kin, pin, pout_hbm, kscr, pscr, kscr2,
               kbuf, pbuf, kob, pob, posb, hist2, cur, hmatbuf, scr, hmat):
    cid = lax.axis_index("c")
    tid = lax.axis_index("s")

    @pl.when(cid == 0)
    def _():
        args = (tid, kbuf, pbuf, kob, pob, posb, hist2, cur, hmatbuf, scr,
                hmat)
        _one_pass(0, True, kin, pin, kscr, pscr, *args)
        _one_pass(8, True, kscr, pscr, kscr2, pout_hbm, *args)
        _one_pass(16, True, kscr2, pout_hbm, kscr, pscr, *args)
        _one_pass(24, False, kscr, pscr, kscr2, pout_hbm, *args)


def _sc_radix_sort(keys, payload):
    mesh = plsc.VectorSubcoreMesh(core_axis_name="c", subcore_axis_name="s")
    f = pl.kernel(
        _sort_body,
        out_type=jax.ShapeDtypeStruct((NUM_EDGE,), jnp.int32),
        mesh=mesh,
        compiler_params=pltpu.CompilerParams(needs_layout_passes=False),
        scratch_types=[
            pltpu.HBM((NUM_EDGE,), jnp.int32),
            pltpu.HBM((NUM_EDGE,), jnp.int32),
            pltpu.HBM((NUM_EDGE,), jnp.int32),
            pltpu.VMEM((_W,), jnp.int32),
            pltpu.VMEM((_W,), jnp.int32),
            pltpu.VMEM((_W,), jnp.int32),
            pltpu.VMEM((_W,), jnp.int32),
            pltpu.VMEM((_W,), jnp.int32),
            pltpu.VMEM((_NB * 16,), jnp.int32),
            pltpu.VMEM((_NB,), jnp.int32),
            pltpu.VMEM((_NT, _NB), jnp.int32),
            pltpu.VMEM((16,), jnp.int32),
            pltpu.VMEM_SHARED((_NT, _NB), jnp.int32),
        ],
    )
    return f(keys, payload)


def kernel(edge_index, edge_weight):
    node_in = edge_index[0].astype(jnp.int32)
    node_out = edge_index[1].astype(jnp.int32)

    degree_in = jax.ops.segment_sum(edge_weight, node_in, num_segments=N_NODES)
    degree_out = jax.ops.segment_sum(edge_weight, node_out, num_segments=N_NODES)

    prob = _sc_prob(1.0 / degree_in, 1.0 / degree_out, node_in, node_out)
    m = jnp.mean(prob)

    u = jax.random.uniform(jax.random.key(42), (NUM_EDGE,), dtype=jnp.float32,
                           minval=1e-20, maxval=1.0)
    gumbel = -jnp.log(-jnp.log(u))

    s, prob_n, key = _scores(prob, m, gumbel)

    perm = _sc_radix_sort(key, jnp.arange(NUM_EDGE, dtype=jnp.int32))
    index = perm[:NUM_SAMPLE]

    new_in = jnp.take(node_in, index)
    new_out = jnp.take(node_out, index)
    new_edge_index = jnp.stack([new_in, new_out]).astype(jnp.int64)
    new_edge_weight = jnp.take(edge_weight, index) / (
        NUM_SAMPLE * jnp.take(prob_n, index) / NUM_EDGE)
    return new_edge_index, new_edge_weight


# speedup vs baseline: 5.9985x; 2.8237x over previous
import functools

import jax
import jax.numpy as jnp
from jax import lax
from jax.experimental import pallas as pl
from jax.experimental.pallas import tpu as pltpu
from jax.experimental.pallas import tpu_sc as plsc

N_NODES = 50000
NUM_EDGE = 1600000
NUM_SAMPLE = 800000

_ROWS = NUM_EDGE // 128

_NT = 16
_CHUNK = NUM_EDGE // _NT
_W = 4000
_NWIN = _CHUNK // _W
_NVREG = _W // 16
_NB = 256
_NPASS = 4


def _score_body(m_ref, prob_ref, gum_ref, s_ref, pn_ref, key_ref):
    m = m_ref[0, 0]
    pn = prob_ref[...] / m
    pn_ref[...] = pn
    s = jnp.log(pn) + gum_ref[...]
    s_ref[...] = s
    b = pltpu.bitcast(s, jnp.int32)
    key_ref[...] = jnp.where(b >= 0, jnp.int32(0x7FFFFFFF) - b, b)


def _scores(prob, m, gumbel):
    prob2 = prob.reshape(_ROWS, 128)
    gum2 = gumbel.reshape(_ROWS, 128)
    m1 = m.reshape(1, 1)
    s, pn, key = pl.pallas_call(
        _score_body,
        out_shape=(
            jax.ShapeDtypeStruct((_ROWS, 128), jnp.float32),
            jax.ShapeDtypeStruct((_ROWS, 128), jnp.float32),
            jax.ShapeDtypeStruct((_ROWS, 128), jnp.int32),
        ),
        in_specs=[
            pl.BlockSpec(memory_space=pltpu.SMEM),
            pl.BlockSpec(memory_space=pltpu.VMEM),
            pl.BlockSpec(memory_space=pltpu.VMEM),
        ],
        out_specs=(
            pl.BlockSpec(memory_space=pltpu.VMEM),
            pl.BlockSpec(memory_space=pltpu.VMEM),
            pl.BlockSpec(memory_space=pltpu.VMEM),
        ),
    )(m1, prob2, gum2)
    return s.reshape(NUM_EDGE), pn.reshape(NUM_EDGE), key.reshape(NUM_EDGE)


_GW = 10000
_GCHUNK = NUM_EDGE // 32
_GNWIN = _GCHUNK // _GW


def _prob_body(rin, rout, nin, nout, prob_hbm,
               nbuf, mbuf, gbuf, hbuf, obuf, sem):
    cid = lax.axis_index("c")
    sid = lax.axis_index("s")
    wid = sid * 2 + cid
    base = wid * jnp.int32(_GCHUNK)

    def add_vreg(j, off):
        obuf[pl.ds(off, 16)] = gbuf[pl.ds(off, 16)] + hbuf[pl.ds(off, 16)]
        return off + 16

    def win(w, off):
        start = pl.multiple_of(base + off, 8)
        pltpu.sync_copy(nin.at[pl.ds(start, _GW)], nbuf)
        pltpu.sync_copy(nout.at[pl.ds(start, _GW)], mbuf)
        cin = pltpu.async_copy(rin.at[nbuf], gbuf, sem)
        cin.wait()
        cout = pltpu.async_copy(rout.at[mbuf], hbuf, sem)
        cout.wait()
        lax.fori_loop(0, _GW // 16, add_vreg, jnp.int32(0))
        pltpu.sync_copy(obuf, prob_hbm.at[pl.ds(start, _GW)])
        return off + _GW

    lax.fori_loop(0, _GNWIN, win, jnp.int32(0))


def _sc_prob(r_in, r_out, node_in, node_out):
    mesh = plsc.VectorSubcoreMesh(core_axis_name="c", subcore_axis_name="s")
    f = pl.kernel(
        _prob_body,
        out_type=jax.ShapeDtypeStruct((NUM_EDGE,), jnp.float32),
        mesh=mesh,
        compiler_params=pltpu.CompilerParams(needs_layout_passes=False),
        scratch_types=[
            pltpu.VMEM((_GW,), jnp.int32),
            pltpu.VMEM((_GW,), jnp.int32),
            pltpu.VMEM((_GW,), jnp.float32),
            pltpu.VMEM((_GW,), jnp.float32),
            pltpu.VMEM((_GW,), jnp.float32),
            pltpu.SemaphoreType.DMA,
        ],
    )
    return f(r_in, r_out, node_in, node_out)


def _one_pass(shift, key0, pin, pout, tid,
              kbuf, pbuf, kob, pob, posb, hist2, cur, hmatbuf, scr, hmat,
              spay, sem):
    lane = lax.iota(jnp.int32, 16)
    base = tid * jnp.int32(_CHUNK)
    shift = jnp.int32(shift)

    zero16 = jnp.zeros((16,), jnp.int32)
    ones16 = jnp.ones((16,), jnp.int32)

    def zero_vreg(j, off):
        hist2[pl.ds(off, 16)] = zero16
        return off + 16

    lax.fori_loop(0, _NB, zero_vreg, jnp.int32(0))

    def hist_vreg(j, off):
        k = kbuf[pl.ds(off, 16)]
        dig = lax.shift_right_logical(k, shift) & 255
        plsc.addupdate_scatter(hist2, [lane * _NB + dig], ones16)
        return off + 16

    def hist_win(w, off):
        pltpu.sync_copy(pin.at[pl.ds(pl.multiple_of(base + off, 8), _W)], pbuf)
        pltpu.async_copy(key0.at[pbuf], kbuf, sem).wait()
        lax.fori_loop(0, _NVREG, hist_vreg, jnp.int32(0))
        return off + _W

    lax.fori_loop(0, _NWIN, hist_win, jnp.int32(0))

    def red_lane(l, c):
        loff, g16, tot = c
        return (loff + _NB, g16, tot + hist2[pl.ds(loff + g16, 16)])

    def red_grp(g, g16):
        _, _, tot = lax.fori_loop(0, 16, red_lane, (jnp.int32(0), g16, zero16))
        kob[pl.ds(g16, 16)] = tot
        return g16 + 16

    lax.fori_loop(0, _NB // 16, red_grp, jnp.int32(0))

    pltpu.sync_copy(kob.at[pl.ds(0, _NB)], hmat.at[tid])
    plsc.subcore_barrier()
    pltpu.sync_copy(hmat, hmatbuf)
    plsc.subcore_barrier()

    def scan_tile(t, c):
        ti, g16, tot, pre = c
        row = hmatbuf[ti, pl.ds(g16, 16)]
        return (ti + 1, g16, tot + row,
                pre + jnp.where(ti < tid, row, zero16))

    def scan_grp(g, c):
        g16, carry = c
        _, _, tot, pre = lax.fori_loop(
            0, _NT, scan_tile, (jnp.int32(0), g16, zero16, zero16))
        cs = plsc.cumsum(tot)
        excl = (cs - tot) + carry
        cur[pl.ds(g16, 16)] = excl + pre
        return (g16 + 16, carry + lax.reduce_max(cs, (0,)))

    lax.fori_loop(0, _NB // 16, scan_grp, (jnp.int32(0), jnp.int32(0)))

    big = jnp.full((16,), 16, jnp.int32)
    lane_m1 = jnp.maximum(lane - 1, 0)
    lane_p1 = jnp.minimum(lane + 1, 15)

    def rank_vreg(j, off):
        k = kbuf[pl.ds(off, 16)]
        p = pbuf[pl.ds(off, 16)]
        dig = lax.shift_right_logical(k, shift) & 255
        ck = dig * 16 + lane
        _, k_s = plsc.sort_key_val(ck, k)
        ck_s, p_s = plsc.sort_key_val(ck, p)
        dig_s = lax.shift_right_logical(ck_s, jnp.int32(4))
        scr[...] = dig_s
        prev = plsc.load_gather(scr, [lane_m1])
        head = (lane == 0) | (dig_s != prev)
        start = plsc.cummax(jnp.where(head, lane, 0))
        rank = lane - start
        c = plsc.load_gather(cur, [dig_s])
        pos = c + rank
        pob[pl.ds(off, 16)] = p_s
        posb[pl.ds(off, 16)] = pos
        plsc.addupdate_scatter(cur, [dig_s], ones16)
        return off + 16

    def rank_win(w, off):
        pltpu.sync_copy(pin.at[pl.ds(pl.multiple_of(base + off, 8), _W)], pbuf)
        pltpu.async_copy(key0.at[pbuf], kbuf, sem).wait()
        lax.fori_loop(0, _NVREG, rank_vreg, jnp.int32(0))
        pltpu.sync_copy(pob, spay.at[posb])
        return off + _W

    lax.fori_loop(0, _NWIN, rank_win, jnp.int32(0))

    plsc.subcore_barrier()

    def back_win(w, off):
        start = pl.multiple_of(base + off, 8)
        pltpu.sync_copy(spay.at[pl.ds(start, _W)], pbuf)
        pltpu.sync_copy(pbuf, pout.at[pl.ds(start, _W)])
        return off + _W

    lax.fori_loop(0, _NWIN, back_win, jnp.int32(0))

    plsc.subcore_barrier()


def _sort_body(kin, pin, pout_hbm,
               kbuf, pbuf, kob, pob, posb, hist2, cur, hmatbuf, scr, hmat,
               spay, sem):
    cid = lax.axis_index("c")
    tid = lax.axis_index("s")

    @pl.when(cid == 0)
    def _():
        args = (tid, kbuf, pbuf, kob, pob, posb, hist2, cur, hmatbuf, scr,
                hmat, spay, sem)
        _one_pass(0, kin, pin, pout_hbm, *args)
        _one_pass(8, kin, pout_hbm, pout_hbm, *args)
        _one_pass(16, kin, pout_hbm, pout_hbm, *args)
        _one_pass(24, kin, pout_hbm, pout_hbm, *args)


def _sc_radix_sort(keys, payload):
    mesh = plsc.VectorSubcoreMesh(core_axis_name="c", subcore_axis_name="s")
    f = pl.kernel(
        _sort_body,
        out_type=jax.ShapeDtypeStruct((NUM_EDGE,), jnp.int32),
        mesh=mesh,
        compiler_params=pltpu.CompilerParams(needs_layout_passes=False),
        scratch_types=[
            pltpu.VMEM((_W,), jnp.int32),
            pltpu.VMEM((_W,), jnp.int32),
            pltpu.VMEM((_NB,), jnp.int32),
            pltpu.VMEM((_W,), jnp.int32),
            pltpu.VMEM((_W,), jnp.int32),
            pltpu.VMEM((_NB * 16,), jnp.int32),
            pltpu.VMEM((_NB,), jnp.int32),
            pltpu.VMEM((_NT, _NB), jnp.int32),
            pltpu.VMEM((16,), jnp.int32),
            pltpu.VMEM_SHARED((_NT, _NB), jnp.int32),
            pltpu.VMEM_SHARED((NUM_EDGE,), jnp.int32),
            pltpu.SemaphoreType.DMA,
        ],
    )
    return f(keys, payload)


def kernel(edge_index, edge_weight):
    node_in = edge_index[0].astype(jnp.int32)
    node_out = edge_index[1].astype(jnp.int32)

    degree_in = jax.ops.segment_sum(edge_weight, node_in, num_segments=N_NODES)
    degree_out = jax.ops.segment_sum(edge_weight, node_out, num_segments=N_NODES)

    prob = _sc_prob(1.0 / degree_in, 1.0 / degree_out, node_in, node_out)
    m = jnp.mean(prob)

    u = jax.random.uniform(jax.random.key(42), (NUM_EDGE,), dtype=jnp.float32,
                           minval=1e-20, maxval=1.0)
    gumbel = -jnp.log(-jnp.log(u))

    s, prob_n, key = _scores(prob, m, gumbel)

    perm = _sc_radix_sort(key, jnp.arange(NUM_EDGE, dtype=jnp.int32))
    index = perm[:NUM_SAMPLE]

    new_in = jnp.take(node_in, index)
    new_out = jnp.take(node_out, index)
    new_edge_index = jnp.stack([new_in, new_out]).astype(jnp.int64)
    new_edge_weight = jnp.take(edge_weight, index) / (
        NUM_SAMPLE * jnp.take(prob_n, index) / NUM_EDGE)
    return new_edge_index, new_edge_weight


# speedup vs baseline: 6.0856x; 1.0145x over previous
import functools

import jax
import jax.numpy as jnp
from jax import lax
from jax.experimental import pallas as pl
from jax.experimental.pallas import tpu as pltpu
from jax.experimental.pallas import tpu_sc as plsc

N_NODES = 50000
NUM_EDGE = 1600000
NUM_SAMPLE = 800000

_ROWS = NUM_EDGE // 128

_NT = 16
_CHUNK = NUM_EDGE // _NT
_W = 4000
_NWIN = _CHUNK // _W
_NVREG = _W // 16
_NB = 256
_NPASS = 4


def _score_body(m_ref, prob_ref, gum_ref, s_ref, pn_ref, key_ref):
    m = m_ref[0, 0]
    pn = prob_ref[...] / m
    pn_ref[...] = pn
    s = jnp.log(pn) + gum_ref[...]
    s_ref[...] = s
    b = pltpu.bitcast(s, jnp.int32)
    key_ref[...] = jnp.where(b >= 0, jnp.int32(0x7FFFFFFF) - b, b)


def _scores(prob, m, gumbel):
    prob2 = prob.reshape(_ROWS, 128)
    gum2 = gumbel.reshape(_ROWS, 128)
    m1 = m.reshape(1, 1)
    s, pn, key = pl.pallas_call(
        _score_body,
        out_shape=(
            jax.ShapeDtypeStruct((_ROWS, 128), jnp.float32),
            jax.ShapeDtypeStruct((_ROWS, 128), jnp.float32),
            jax.ShapeDtypeStruct((_ROWS, 128), jnp.int32),
        ),
        in_specs=[
            pl.BlockSpec(memory_space=pltpu.SMEM),
            pl.BlockSpec(memory_space=pltpu.VMEM),
            pl.BlockSpec(memory_space=pltpu.VMEM),
        ],
        out_specs=(
            pl.BlockSpec(memory_space=pltpu.VMEM),
            pl.BlockSpec(memory_space=pltpu.VMEM),
            pl.BlockSpec(memory_space=pltpu.VMEM),
        ),
    )(m1, prob2, gum2)
    return s.reshape(NUM_EDGE), pn.reshape(NUM_EDGE), key.reshape(NUM_EDGE)


_GW = 10000
_GCHUNK = NUM_EDGE // 32
_GNWIN = _GCHUNK // _GW


def _prob_body(rin, rout, nin, nout, prob_hbm,
               nbuf, mbuf, gbuf, hbuf, obuf, sem):
    cid = lax.axis_index("c")
    sid = lax.axis_index("s")
    wid = sid * 2 + cid
    base = wid * jnp.int32(_GCHUNK)

    def add_vreg(j, off):
        obuf[pl.ds(off, 16)] = gbuf[pl.ds(off, 16)] + hbuf[pl.ds(off, 16)]
        return off + 16

    def win(w, off):
        start = pl.multiple_of(base + off, 8)
        pltpu.sync_copy(nin.at[pl.ds(start, _GW)], nbuf)
        pltpu.sync_copy(nout.at[pl.ds(start, _GW)], mbuf)
        cin = pltpu.async_copy(rin.at[nbuf], gbuf, sem)
        cin.wait()
        cout = pltpu.async_copy(rout.at[mbuf], hbuf, sem)
        cout.wait()
        lax.fori_loop(0, _GW // 16, add_vreg, jnp.int32(0))
        pltpu.sync_copy(obuf, prob_hbm.at[pl.ds(start, _GW)])
        return off + _GW

    lax.fori_loop(0, _GNWIN, win, jnp.int32(0))


def _sc_prob(r_in, r_out, node_in, node_out):
    mesh = plsc.VectorSubcoreMesh(core_axis_name="c", subcore_axis_name="s")
    f = pl.kernel(
        _prob_body,
        out_type=jax.ShapeDtypeStruct((NUM_EDGE,), jnp.float32),
        mesh=mesh,
        compiler_params=pltpu.CompilerParams(needs_layout_passes=False),
        scratch_types=[
            pltpu.VMEM((_GW,), jnp.int32),
            pltpu.VMEM((_GW,), jnp.int32),
            pltpu.VMEM((_GW,), jnp.float32),
            pltpu.VMEM((_GW,), jnp.float32),
            pltpu.VMEM((_GW,), jnp.float32),
            pltpu.SemaphoreType.DMA,
        ],
    )
    return f(r_in, r_out, node_in, node_out)


_SCHUNK = NUM_SAMPLE // 32


def _sel_body(nin, nout, pn, wts, idx, oin, oout, opn, owt,
              ibuf, b0, b1, b2, b3, sem):
    cid = lax.axis_index("c")
    sid = lax.axis_index("s")
    wid = sid * 2 + cid
    start = pl.multiple_of(wid * jnp.int32(_SCHUNK), 8)
    sl = pl.ds(start, _SCHUNK)
    pltpu.sync_copy(idx.at[sl], ibuf)
    pltpu.async_copy(nin.at[ibuf], b0, sem).wait()
    pltpu.async_copy(nout.at[ibuf], b1, sem).wait()
    pltpu.async_copy(pn.at[ibuf], b2, sem).wait()
    pltpu.async_copy(wts.at[ibuf], b3, sem).wait()
    pltpu.sync_copy(b0, oin.at[sl])
    pltpu.sync_copy(b1, oout.at[sl])
    pltpu.sync_copy(b2, opn.at[sl])
    pltpu.sync_copy(b3, owt.at[sl])


def _sc_select(node_in, node_out, prob_n, weights, index):
    mesh = plsc.VectorSubcoreMesh(core_axis_name="c", subcore_axis_name="s")
    f = pl.kernel(
        _sel_body,
        out_type=(
            jax.ShapeDtypeStruct((NUM_SAMPLE,), jnp.int32),
            jax.ShapeDtypeStruct((NUM_SAMPLE,), jnp.int32),
            jax.ShapeDtypeStruct((NUM_SAMPLE,), jnp.float32),
            jax.ShapeDtypeStruct((NUM_SAMPLE,), jnp.float32),
        ),
        mesh=mesh,
        compiler_params=pltpu.CompilerParams(needs_layout_passes=False),
        scratch_types=[
            pltpu.VMEM((_SCHUNK,), jnp.int32),
            pltpu.VMEM((_SCHUNK,), jnp.int32),
            pltpu.VMEM((_SCHUNK,), jnp.int32),
            pltpu.VMEM((_SCHUNK,), jnp.float32),
            pltpu.VMEM((_SCHUNK,), jnp.float32),
            pltpu.SemaphoreType.DMA,
        ],
    )
    return f(node_in, node_out, prob_n, weights, index)


def _one_pass(shift, key0, pin, pout, tid,
              kbuf, pbuf, kob, pob, posb, hist2, cur, hmatbuf, scr, hmat,
              spay, sem):
    lane = lax.iota(jnp.int32, 16)
    base = tid * jnp.int32(_CHUNK)
    shift = jnp.int32(shift)

    zero16 = jnp.zeros((16,), jnp.int32)
    ones16 = jnp.ones((16,), jnp.int32)

    def zero_vreg(j, off):
        hist2[pl.ds(off, 16)] = zero16
        return off + 16

    lax.fori_loop(0, _NB, zero_vreg, jnp.int32(0))

    def hist_vreg(j, off):
        k = kbuf[pl.ds(off, 16)]
        dig = lax.shift_right_logical(k, shift) & 255
        plsc.addupdate_scatter(hist2, [lane * _NB + dig], ones16)
        return off + 16

    def hist_win(w, off):
        pltpu.sync_copy(pin.at[pl.ds(pl.multiple_of(base + off, 8), _W)], pbuf)
        pltpu.async_copy(key0.at[pbuf], kbuf, sem).wait()
        lax.fori_loop(0, _NVREG, hist_vreg, jnp.int32(0))
        return off + _W

    lax.fori_loop(0, _NWIN, hist_win, jnp.int32(0))

    def red_lane(l, c):
        loff, g16, tot = c
        return (loff + _NB, g16, tot + hist2[pl.ds(loff + g16, 16)])

    def red_grp(g, g16):
        _, _, tot = lax.fori_loop(0, 16, red_lane, (jnp.int32(0), g16, zero16))
        kob[pl.ds(g16, 16)] = tot
        return g16 + 16

    lax.fori_loop(0, _NB // 16, red_grp, jnp.int32(0))

    pltpu.sync_copy(kob.at[pl.ds(0, _NB)], hmat.at[tid])
    plsc.subcore_barrier()
    pltpu.sync_copy(hmat, hmatbuf)
    plsc.subcore_barrier()

    def scan_tile(t, c):
        ti, g16, tot, pre = c
        row = hmatbuf[ti, pl.ds(g16, 16)]
        return (ti + 1, g16, tot + row,
                pre + jnp.where(ti < tid, row, zero16))

    def scan_grp(g, c):
        g16, carry = c
        _, _, tot, pre = lax.fori_loop(
            0, _NT, scan_tile, (jnp.int32(0), g16, zero16, zero16))
        cs = plsc.cumsum(tot)
        excl = (cs - tot) + carry
        cur[pl.ds(g16, 16)] = excl + pre
        return (g16 + 16, carry + lax.reduce_max(cs, (0,)))

    lax.fori_loop(0, _NB // 16, scan_grp, (jnp.int32(0), jnp.int32(0)))

    big = jnp.full((16,), 16, jnp.int32)
    lane_m1 = jnp.maximum(lane - 1, 0)
    lane_p1 = jnp.minimum(lane + 1, 15)

    def rank_vreg(j, off):
        k = kbuf[pl.ds(off, 16)]
        p = pbuf[pl.ds(off, 16)]
        dig = lax.shift_right_logical(k, shift) & 255
        ck = dig * 16 + lane
        _, k_s = plsc.sort_key_val(ck, k)
        ck_s, p_s = plsc.sort_key_val(ck, p)
        dig_s = lax.shift_right_logical(ck_s, jnp.int32(4))
        scr[...] = dig_s
        prev = plsc.load_gather(scr, [lane_m1])
        head = (lane == 0) | (dig_s != prev)
        start = plsc.cummax(jnp.where(head, lane, 0))
        rank = lane - start
        c = plsc.load_gather(cur, [dig_s])
        pos = c + rank
        pob[pl.ds(off, 16)] = p_s
        posb[pl.ds(off, 16)] = pos
        plsc.addupdate_scatter(cur, [dig_s], ones16)
        return off + 16

    def rank_win(w, off):
        pltpu.sync_copy(pin.at[pl.ds(pl.multiple_of(base + off, 8), _W)], pbuf)
        pltpu.async_copy(key0.at[pbuf], kbuf, sem).wait()
        lax.fori_loop(0, _NVREG, rank_vreg, jnp.int32(0))
        pltpu.sync_copy(pob, spay.at[posb])
        return off + _W

    lax.fori_loop(0, _NWIN, rank_win, jnp.int32(0))

    plsc.subcore_barrier()

    def back_win(w, off):
        start = pl.multiple_of(base + off, 8)
        pltpu.sync_copy(spay.at[pl.ds(start, _W)], pbuf)
        pltpu.sync_copy(pbuf, pout.at[pl.ds(start, _W)])
        return off + _W

    lax.fori_loop(0, _NWIN, back_win, jnp.int32(0))

    plsc.subcore_barrier()


def _sort_body(kin, pin, pout_hbm,
               kbuf, pbuf, kob, pob, posb, hist2, cur, hmatbuf, scr, hmat,
               spay, sem):
    cid = lax.axis_index("c")
    tid = lax.axis_index("s")

    @pl.when(cid == 0)
    def _():
        args = (tid, kbuf, pbuf, kob, pob, posb, hist2, cur, hmatbuf, scr,
                hmat, spay, sem)
        _one_pass(0, kin, pin, pout_hbm, *args)
        _one_pass(8, kin, pout_hbm, pout_hbm, *args)
        _one_pass(16, kin, pout_hbm, pout_hbm, *args)
        _one_pass(24, kin, pout_hbm, pout_hbm, *args)


def _sc_radix_sort(keys, payload):
    mesh = plsc.VectorSubcoreMesh(core_axis_name="c", subcore_axis_name="s")
    f = pl.kernel(
        _sort_body,
        out_type=jax.ShapeDtypeStruct((NUM_EDGE,), jnp.int32),
        mesh=mesh,
        compiler_params=pltpu.CompilerParams(needs_layout_passes=False),
        scratch_types=[
            pltpu.VMEM((_W,), jnp.int32),
            pltpu.VMEM((_W,), jnp.int32),
            pltpu.VMEM((_NB,), jnp.int32),
            pltpu.VMEM((_W,), jnp.int32),
            pltpu.VMEM((_W,), jnp.int32),
            pltpu.VMEM((_NB * 16,), jnp.int32),
            pltpu.VMEM((_NB,), jnp.int32),
            pltpu.VMEM((_NT, _NB), jnp.int32),
            pltpu.VMEM((16,), jnp.int32),
            pltpu.VMEM_SHARED((_NT, _NB), jnp.int32),
            pltpu.VMEM_SHARED((NUM_EDGE,), jnp.int32),
            pltpu.SemaphoreType.DMA,
        ],
    )
    return f(keys, payload)


def kernel(edge_index, edge_weight):
    node_in = edge_index[0].astype(jnp.int32)
    node_out = edge_index[1].astype(jnp.int32)

    degree_in = jax.ops.segment_sum(edge_weight, node_in, num_segments=N_NODES)
    degree_out = jax.ops.segment_sum(edge_weight, node_out, num_segments=N_NODES)

    prob = _sc_prob(1.0 / degree_in, 1.0 / degree_out, node_in, node_out)
    m = jnp.mean(prob)

    u = jax.random.uniform(jax.random.key(42), (NUM_EDGE,), dtype=jnp.float32,
                           minval=1e-20, maxval=1.0)
    gumbel = -jnp.log(-jnp.log(u))

    s, prob_n, key = _scores(prob, m, gumbel)

    perm = _sc_radix_sort(key, jnp.arange(NUM_EDGE, dtype=jnp.int32))
    index = perm[:NUM_SAMPLE]

    new_in, new_out, sel_p, sel_w = _sc_select(
        node_in, node_out, prob_n, edge_weight, index)
    new_edge_index = jnp.stack([new_in, new_out]).astype(jnp.int64)
    new_edge_weight = sel_w / (NUM_SAMPLE * sel_p / NUM_EDGE)
    return new_edge_index, new_edge_weight


# speedup vs baseline: 12.4703x; 2.0491x over previous
import functools

import jax
import jax.numpy as jnp
from jax import lax
from jax.experimental import pallas as pl
from jax.experimental.pallas import tpu as pltpu
from jax.experimental.pallas import tpu_sc as plsc

N_NODES = 50000
NUM_EDGE = 1600000
NUM_SAMPLE = 800000

_ROWS = NUM_EDGE // 128

_NT = 16
_CHUNK = NUM_EDGE // _NT
_W = 4000
_NWIN = _CHUNK // _W
_NVREG = _W // 16
_NB = 256
_NPASS = 4


def _score_body(m_ref, prob_ref, gum_ref, s_ref, pn_ref, key_ref):
    m = m_ref[0, 0]
    pn = prob_ref[...] / m
    pn_ref[...] = pn
    s = jnp.log(pn) + gum_ref[...]
    s_ref[...] = s
    b = pltpu.bitcast(s, jnp.int32)
    key_ref[...] = jnp.where(b >= 0, jnp.int32(0x7FFFFFFF) - b, b)


def _scores(prob, m, gumbel):
    prob2 = prob.reshape(_ROWS, 128)
    gum2 = gumbel.reshape(_ROWS, 128)
    m1 = m.reshape(1, 1)
    s, pn, key = pl.pallas_call(
        _score_body,
        out_shape=(
            jax.ShapeDtypeStruct((_ROWS, 128), jnp.float32),
            jax.ShapeDtypeStruct((_ROWS, 128), jnp.float32),
            jax.ShapeDtypeStruct((_ROWS, 128), jnp.int32),
        ),
        in_specs=[
            pl.BlockSpec(memory_space=pltpu.SMEM),
            pl.BlockSpec(memory_space=pltpu.VMEM),
            pl.BlockSpec(memory_space=pltpu.VMEM),
        ],
        out_specs=(
            pl.BlockSpec(memory_space=pltpu.VMEM),
            pl.BlockSpec(memory_space=pltpu.VMEM),
            pl.BlockSpec(memory_space=pltpu.VMEM),
        ),
    )(m1, prob2, gum2)
    return s.reshape(NUM_EDGE), pn.reshape(NUM_EDGE), key.reshape(NUM_EDGE)


_GW = 10000
_GCHUNK = NUM_EDGE // 32
_GNWIN = _GCHUNK // _GW


def _prob_body(rin, rout, nin, nout, prob_hbm,
               nbuf, mbuf, gbuf, hbuf, obuf, sem):
    cid = lax.axis_index("c")
    sid = lax.axis_index("s")
    wid = sid * 2 + cid
    base = wid * jnp.int32(_GCHUNK)

    def add_vreg(j, off):
        obuf[pl.ds(off, 16)] = gbuf[pl.ds(off, 16)] + hbuf[pl.ds(off, 16)]
        return off + 16

    def win(w, off):
        start = pl.multiple_of(base + off, 8)
        pltpu.sync_copy(nin.at[pl.ds(start, _GW)], nbuf)
        pltpu.sync_copy(nout.at[pl.ds(start, _GW)], mbuf)
        cin = pltpu.async_copy(rin.at[nbuf], gbuf, sem)
        cin.wait()
        cout = pltpu.async_copy(rout.at[mbuf], hbuf, sem)
        cout.wait()
        lax.fori_loop(0, _GW // 16, add_vreg, jnp.int32(0))
        pltpu.sync_copy(obuf, prob_hbm.at[pl.ds(start, _GW)])
        return off + _GW

    lax.fori_loop(0, _GNWIN, win, jnp.int32(0))


def _sc_prob(r_in, r_out, node_in, node_out):
    mesh = plsc.VectorSubcoreMesh(core_axis_name="c", subcore_axis_name="s")
    f = pl.kernel(
        _prob_body,
        out_type=jax.ShapeDtypeStruct((NUM_EDGE,), jnp.float32),
        mesh=mesh,
        compiler_params=pltpu.CompilerParams(needs_layout_passes=False),
        scratch_types=[
            pltpu.VMEM((_GW,), jnp.int32),
            pltpu.VMEM((_GW,), jnp.int32),
            pltpu.VMEM((_GW,), jnp.float32),
            pltpu.VMEM((_GW,), jnp.float32),
            pltpu.VMEM((_GW,), jnp.float32),
            pltpu.SemaphoreType.DMA,
        ],
    )
    return f(r_in, r_out, node_in, node_out)


_DPAD = 50048
_DSL = _DPAD // 16
_DW = 10000


def _deg_body(nin, nout, wts, din, dout, ibuf, vbuf, spdeg, sem):
    cid = lax.axis_index("c")
    sid = lax.axis_index("s")
    base = sid * jnp.int32(_CHUNK)
    zero16 = jnp.zeros((16,), jnp.float32)

    def zfill(j, off):
        vbuf[pl.ds(off, 16)] = zero16
        return off + 16

    lax.fori_loop(0, _DSL // 16 + 1, zfill, jnp.int32(0))
    zsl = pl.ds(pl.multiple_of(sid * jnp.int32(_DSL), 8), _DSL)
    pltpu.sync_copy(vbuf.at[pl.ds(0, _DSL)], spdeg.at[zsl])
    plsc.subcore_barrier()

    def win(w, off):
        sl = pl.ds(pl.multiple_of(base + off, 8), _DW)
        pltpu.sync_copy(wts.at[sl], vbuf)

        @pl.when(cid == 0)
        def _():
            pltpu.sync_copy(nin.at[sl], ibuf)

        @pl.when(cid == 1)
        def _():
            pltpu.sync_copy(nout.at[sl], ibuf)

        pltpu.sync_copy(vbuf, spdeg.at[ibuf], add=True)
        return off + _DW

    lax.fori_loop(0, _CHUNK // _DW, win, jnp.int32(0))
    plsc.subcore_barrier()

    @pl.when(cid == 0)
    def _():
        pltpu.sync_copy(spdeg.at[zsl], vbuf.at[pl.ds(0, _DSL)])
        pltpu.sync_copy(vbuf.at[pl.ds(0, _DSL)], din.at[zsl])

    @pl.when(cid == 1)
    def _():
        pltpu.sync_copy(spdeg.at[zsl], vbuf.at[pl.ds(0, _DSL)])
        pltpu.sync_copy(vbuf.at[pl.ds(0, _DSL)], dout.at[zsl])


def _sc_degrees(node_in, node_out, weights):
    mesh = plsc.VectorSubcoreMesh(core_axis_name="c", subcore_axis_name="s")
    f = pl.kernel(
        _deg_body,
        out_type=(
            jax.ShapeDtypeStruct((_DPAD,), jnp.float32),
            jax.ShapeDtypeStruct((_DPAD,), jnp.float32),
        ),
        mesh=mesh,
        compiler_params=pltpu.CompilerParams(needs_layout_passes=False),
        scratch_types=[
            pltpu.VMEM((_DW,), jnp.int32),
            pltpu.VMEM((_DW,), jnp.float32),
            pltpu.VMEM_SHARED((_DPAD,), jnp.float32),
            pltpu.SemaphoreType.DMA,
        ],
    )
    din, dout = f(node_in, node_out, weights)
    return din[:N_NODES], dout[:N_NODES]


_SCHUNK = NUM_SAMPLE // 32


def _sel_body(nin, nout, pn, wts, idx, oin, oout, opn, owt,
              ibuf, b0, b1, b2, b3, sem):
    cid = lax.axis_index("c")
    sid = lax.axis_index("s")
    wid = sid * 2 + cid
    start = pl.multiple_of(wid * jnp.int32(_SCHUNK), 8)
    sl = pl.ds(start, _SCHUNK)
    pltpu.sync_copy(idx.at[sl], ibuf)
    pltpu.async_copy(nin.at[ibuf], b0, sem).wait()
    pltpu.async_copy(nout.at[ibuf], b1, sem).wait()
    pltpu.async_copy(pn.at[ibuf], b2, sem).wait()
    pltpu.async_copy(wts.at[ibuf], b3, sem).wait()
    pltpu.sync_copy(b0, oin.at[sl])
    pltpu.sync_copy(b1, oout.at[sl])
    pltpu.sync_copy(b2, opn.at[sl])
    pltpu.sync_copy(b3, owt.at[sl])


def _sc_select(node_in, node_out, prob_n, weights, index):
    mesh = plsc.VectorSubcoreMesh(core_axis_name="c", subcore_axis_name="s")
    f = pl.kernel(
        _sel_body,
        out_type=(
            jax.ShapeDtypeStruct((NUM_SAMPLE,), jnp.int32),
            jax.ShapeDtypeStruct((NUM_SAMPLE,), jnp.int32),
            jax.ShapeDtypeStruct((NUM_SAMPLE,), jnp.float32),
            jax.ShapeDtypeStruct((NUM_SAMPLE,), jnp.float32),
        ),
        mesh=mesh,
        compiler_params=pltpu.CompilerParams(needs_layout_passes=False),
        scratch_types=[
            pltpu.VMEM((_SCHUNK,), jnp.int32),
            pltpu.VMEM((_SCHUNK,), jnp.int32),
            pltpu.VMEM((_SCHUNK,), jnp.int32),
            pltpu.VMEM((_SCHUNK,), jnp.float32),
            pltpu.VMEM((_SCHUNK,), jnp.float32),
            pltpu.SemaphoreType.DMA,
        ],
    )
    return f(node_in, node_out, prob_n, weights, index)


def _one_pass(shift, key0, pin, pout, tid,
              kbuf, pbuf, kob, pob, posb, hist2, cur, hmatbuf, scr, hmat,
              spay, sem):
    lane = lax.iota(jnp.int32, 16)
    base = tid * jnp.int32(_CHUNK)
    shift = jnp.int32(shift)

    zero16 = jnp.zeros((16,), jnp.int32)
    ones16 = jnp.ones((16,), jnp.int32)

    def zero_vreg(j, off):
        hist2[pl.ds(off, 16)] = zero16
        return off + 16

    lax.fori_loop(0, _NB, zero_vreg, jnp.int32(0))

    def hist_vreg(j, off):
        k = kbuf[pl.ds(off, 16)]
        dig = lax.shift_right_logical(k, shift) & 255
        plsc.addupdate_scatter(hist2, [lane * _NB + dig], ones16)
        return off + 16

    def hist_win(w, off):
        pltpu.sync_copy(pin.at[pl.ds(pl.multiple_of(base + off, 8), _W)], pbuf)
        pltpu.async_copy(key0.at[pbuf], kbuf, sem).wait()
        lax.fori_loop(0, _NVREG, hist_vreg, jnp.int32(0))
        return off + _W

    lax.fori_loop(0, _NWIN, hist_win, jnp.int32(0))

    def red_lane(l, c):
        loff, g16, tot = c
        return (loff + _NB, g16, tot + hist2[pl.ds(loff + g16, 16)])

    def red_grp(g, g16):
        _, _, tot = lax.fori_loop(0, 16, red_lane, (jnp.int32(0), g16, zero16))
        kob[pl.ds(g16, 16)] = tot
        return g16 + 16

    lax.fori_loop(0, _NB // 16, red_grp, jnp.int32(0))

    pltpu.sync_copy(kob.at[pl.ds(0, _NB)], hmat.at[tid])
    plsc.subcore_barrier()
    pltpu.sync_copy(hmat, hmatbuf)
    plsc.subcore_barrier()

    def scan_tile(t, c):
        ti, g16, tot, pre = c
        row = hmatbuf[ti, pl.ds(g16, 16)]
        return (ti + 1, g16, tot + row,
                pre + jnp.where(ti < tid, row, zero16))

    def scan_grp(g, c):
        g16, carry = c
        _, _, tot, pre = lax.fori_loop(
            0, _NT, scan_tile, (jnp.int32(0), g16, zero16, zero16))
        cs = plsc.cumsum(tot)
        excl = (cs - tot) + carry
        cur[pl.ds(g16, 16)] = excl + pre
        return (g16 + 16, carry + lax.reduce_max(cs, (0,)))

    lax.fori_loop(0, _NB // 16, scan_grp, (jnp.int32(0), jnp.int32(0)))

    big = jnp.full((16,), 16, jnp.int32)
    lane_m1 = jnp.maximum(lane - 1, 0)
    lane_p1 = jnp.minimum(lane + 1, 15)

    def rank_vreg(j, off):
        k = kbuf[pl.ds(off, 16)]
        p = pbuf[pl.ds(off, 16)]
        dig = lax.shift_right_logical(k, shift) & 255
        ck = dig * 16 + lane
        _, k_s = plsc.sort_key_val(ck, k)
        ck_s, p_s = plsc.sort_key_val(ck, p)
        dig_s = lax.shift_right_logical(ck_s, jnp.int32(4))
        scr[...] = dig_s
        prev = plsc.load_gather(scr, [lane_m1])
        head = (lane == 0) | (dig_s != prev)
        start = plsc.cummax(jnp.where(head, lane, 0))
        rank = lane - start
        c = plsc.load_gather(cur, [dig_s])
        pos = c + rank
        pob[pl.ds(off, 16)] = p_s
        posb[pl.ds(off, 16)] = pos
        plsc.addupdate_scatter(cur, [dig_s], ones16)
        return off + 16

    def rank_win(w, off):
        pltpu.sync_copy(pin.at[pl.ds(pl.multiple_of(base + off, 8), _W)], pbuf)
        pltpu.async_copy(key0.at[pbuf], kbuf, sem).wait()
        lax.fori_loop(0, _NVREG, rank_vreg, jnp.int32(0))
        pltpu.sync_copy(pob, spay.at[posb])
        return off + _W

    lax.fori_loop(0, _NWIN, rank_win, jnp.int32(0))

    plsc.subcore_barrier()

    def back_win(w, off):
        start = pl.multiple_of(base + off, 8)
        pltpu.sync_copy(spay.at[pl.ds(start, _W)], pbuf)
        pltpu.sync_copy(pbuf, pout.at[pl.ds(start, _W)])
        return off + _W

    lax.fori_loop(0, _NWIN, back_win, jnp.int32(0))

    plsc.subcore_barrier()


def _sort_body(kin, pin, pout_hbm,
               kbuf, pbuf, kob, pob, posb, hist2, cur, hmatbuf, scr, hmat,
               spay, sem):
    cid = lax.axis_index("c")
    tid = lax.axis_index("s")

    @pl.when(cid == 0)
    def _():
        args = (tid, kbuf, pbuf, kob, pob, posb, hist2, cur, hmatbuf, scr,
                hmat, spay, sem)
        _one_pass(0, kin, pin, pout_hbm, *args)
        _one_pass(8, kin, pout_hbm, pout_hbm, *args)
        _one_pass(16, kin, pout_hbm, pout_hbm, *args)
        _one_pass(24, kin, pout_hbm, pout_hbm, *args)


def _sc_radix_sort(keys, payload):
    mesh = plsc.VectorSubcoreMesh(core_axis_name="c", subcore_axis_name="s")
    f = pl.kernel(
        _sort_body,
        out_type=jax.ShapeDtypeStruct((NUM_EDGE,), jnp.int32),
        mesh=mesh,
        compiler_params=pltpu.CompilerParams(needs_layout_passes=False),
        scratch_types=[
            pltpu.VMEM((_W,), jnp.int32),
            pltpu.VMEM((_W,), jnp.int32),
            pltpu.VMEM((_NB,), jnp.int32),
            pltpu.VMEM((_W,), jnp.int32),
            pltpu.VMEM((_W,), jnp.int32),
            pltpu.VMEM((_NB * 16,), jnp.int32),
            pltpu.VMEM((_NB,), jnp.int32),
            pltpu.VMEM((_NT, _NB), jnp.int32),
            pltpu.VMEM((16,), jnp.int32),
            pltpu.VMEM_SHARED((_NT, _NB), jnp.int32),
            pltpu.VMEM_SHARED((NUM_EDGE,), jnp.int32),
            pltpu.SemaphoreType.DMA,
        ],
    )
    return f(keys, payload)


def kernel(edge_index, edge_weight):
    node_in = edge_index[0].astype(jnp.int32)
    node_out = edge_index[1].astype(jnp.int32)

    degree_in, degree_out = _sc_degrees(node_in, node_out, edge_weight)

    prob = _sc_prob(1.0 / degree_in, 1.0 / degree_out, node_in, node_out)
    m = jnp.mean(prob)

    u = jax.random.uniform(jax.random.key(42), (NUM_EDGE,), dtype=jnp.float32,
                           minval=1e-20, maxval=1.0)
    gumbel = -jnp.log(-jnp.log(u))

    s, prob_n, key = _scores(prob, m, gumbel)

    perm = _sc_radix_sort(key, jnp.arange(NUM_EDGE, dtype=jnp.int32))
    index = perm[:NUM_SAMPLE]

    new_in, new_out, sel_p, sel_w = _sc_select(
        node_in, node_out, prob_n, edge_weight, index)
    new_edge_index = jnp.stack([new_in, new_out]).astype(jnp.int64)
    new_edge_weight = sel_w / (NUM_SAMPLE * sel_p / NUM_EDGE)
    return new_edge_index, new_edge_weight


# speedup vs baseline: 12.4724x; 1.0002x over previous
import jax
import jax.numpy as jnp
from jax import lax
from jax.experimental import pallas as pl
from jax.experimental.pallas import tpu as pltpu
from jax.experimental.pallas import tpu_sc as plsc

N_NODES = 50000
NUM_EDGE = 1600000
NUM_SAMPLE = 800000

_ROWS = NUM_EDGE // 128

_NT = 16
_CHUNK = NUM_EDGE // _NT
_W = 4000
_NWIN = _CHUNK // _W
_NVREG = _W // 16
_NB = 256


def _score_body(m_ref, prob_ref, gum_ref, s_ref, pn_ref, key_ref):
    m = m_ref[0, 0]
    pn = prob_ref[...] / m
    pn_ref[...] = pn
    s = jnp.log(pn) + gum_ref[...]
    s_ref[...] = s
    b = pltpu.bitcast(s, jnp.int32)
    key_ref[...] = jnp.where(b >= 0, jnp.int32(0x7FFFFFFF) - b, b)


def _scores(prob, m, gumbel):
    prob2 = prob.reshape(_ROWS, 128)
    gum2 = gumbel.reshape(_ROWS, 128)
    m1 = m.reshape(1, 1)
    s, pn, key = pl.pallas_call(
        _score_body,
        out_shape=(
            jax.ShapeDtypeStruct((_ROWS, 128), jnp.float32),
            jax.ShapeDtypeStruct((_ROWS, 128), jnp.float32),
            jax.ShapeDtypeStruct((_ROWS, 128), jnp.int32),
        ),
        in_specs=[
            pl.BlockSpec(memory_space=pltpu.SMEM),
            pl.BlockSpec(memory_space=pltpu.VMEM),
            pl.BlockSpec(memory_space=pltpu.VMEM),
        ],
        out_specs=(
            pl.BlockSpec(memory_space=pltpu.VMEM),
            pl.BlockSpec(memory_space=pltpu.VMEM),
            pl.BlockSpec(memory_space=pltpu.VMEM),
        ),
    )(m1, prob2, gum2)
    return s.reshape(NUM_EDGE), pn.reshape(NUM_EDGE), key.reshape(NUM_EDGE)


_GW = 10000
_GCHUNK = NUM_EDGE // 32
_GNWIN = _GCHUNK // _GW


def _prob_body(rin, rout, nin, nout, prob_hbm,
               nbuf, mbuf, gbuf, hbuf, obuf, sem):
    cid = lax.axis_index("c")
    sid = lax.axis_index("s")
    wid = sid * 2 + cid
    base = wid * jnp.int32(_GCHUNK)

    def add_vreg(j, off):
        obuf[pl.ds(off, 16)] = gbuf[pl.ds(off, 16)] + hbuf[pl.ds(off, 16)]
        return off + 16

    def win(w, off):
        start = pl.multiple_of(base + off, 8)
        pltpu.sync_copy(nin.at[pl.ds(start, _GW)], nbuf)
        pltpu.sync_copy(nout.at[pl.ds(start, _GW)], mbuf)
        cin = pltpu.async_copy(rin.at[nbuf], gbuf, sem)
        cin.wait()
        cout = pltpu.async_copy(rout.at[mbuf], hbuf, sem)
        cout.wait()
        lax.fori_loop(0, _GW // 16, add_vreg, jnp.int32(0))
        pltpu.sync_copy(obuf, prob_hbm.at[pl.ds(start, _GW)])
        return off + _GW

    lax.fori_loop(0, _GNWIN, win, jnp.int32(0))


def _sc_prob(r_in, r_out, node_in, node_out):
    mesh = plsc.VectorSubcoreMesh(core_axis_name="c", subcore_axis_name="s")
    f = pl.kernel(
        _prob_body,
        out_type=jax.ShapeDtypeStruct((NUM_EDGE,), jnp.float32),
        mesh=mesh,
        compiler_params=pltpu.CompilerParams(needs_layout_passes=False),
        scratch_types=[
            pltpu.VMEM((_GW,), jnp.int32),
            pltpu.VMEM((_GW,), jnp.int32),
            pltpu.VMEM((_GW,), jnp.float32),
            pltpu.VMEM((_GW,), jnp.float32),
            pltpu.VMEM((_GW,), jnp.float32),
            pltpu.SemaphoreType.DMA,
        ],
    )
    return f(r_in, r_out, node_in, node_out)


_DPAD = 50048
_DSL = _DPAD // 16
_DW = 10000


def _deg_body(nin, nout, wts, din, dout, ibuf, vbuf, spdeg, sem):
    cid = lax.axis_index("c")
    sid = lax.axis_index("s")
    base = sid * jnp.int32(_CHUNK)
    zero16 = jnp.zeros((16,), jnp.float32)

    def zfill(j, off):
        vbuf[pl.ds(off, 16)] = zero16
        return off + 16

    lax.fori_loop(0, _DSL // 16 + 1, zfill, jnp.int32(0))
    zsl = pl.ds(pl.multiple_of(sid * jnp.int32(_DSL), 8), _DSL)
    pltpu.sync_copy(vbuf.at[pl.ds(0, _DSL)], spdeg.at[zsl])
    plsc.subcore_barrier()

    def win(w, off):
        sl = pl.ds(pl.multiple_of(base + off, 8), _DW)
        pltpu.sync_copy(wts.at[sl], vbuf)

        @pl.when(cid == 0)
        def _():
            pltpu.sync_copy(nin.at[sl], ibuf)

        @pl.when(cid == 1)
        def _():
            pltpu.sync_copy(nout.at[sl], ibuf)

        pltpu.sync_copy(vbuf, spdeg.at[ibuf], add=True)
        return off + _DW

    lax.fori_loop(0, _CHUNK // _DW, win, jnp.int32(0))
    plsc.subcore_barrier()

    @pl.when(cid == 0)
    def _():
        pltpu.sync_copy(spdeg.at[zsl], vbuf.at[pl.ds(0, _DSL)])
        pltpu.sync_copy(vbuf.at[pl.ds(0, _DSL)], din.at[zsl])

    @pl.when(cid == 1)
    def _():
        pltpu.sync_copy(spdeg.at[zsl], vbuf.at[pl.ds(0, _DSL)])
        pltpu.sync_copy(vbuf.at[pl.ds(0, _DSL)], dout.at[zsl])


def _sc_degrees(node_in, node_out, weights):
    mesh = plsc.VectorSubcoreMesh(core_axis_name="c", subcore_axis_name="s")
    f = pl.kernel(
        _deg_body,
        out_type=(
            jax.ShapeDtypeStruct((_DPAD,), jnp.float32),
            jax.ShapeDtypeStruct((_DPAD,), jnp.float32),
        ),
        mesh=mesh,
        compiler_params=pltpu.CompilerParams(needs_layout_passes=False),
        scratch_types=[
            pltpu.VMEM((_DW,), jnp.int32),
            pltpu.VMEM((_DW,), jnp.float32),
            pltpu.VMEM_SHARED((_DPAD,), jnp.float32),
            pltpu.SemaphoreType.DMA,
        ],
    )
    din, dout = f(node_in, node_out, weights)
    return din[:N_NODES], dout[:N_NODES]


_SCHUNK = NUM_SAMPLE // 32


def _sel_body(nin, nout, pn, wts, idx, oin, oout, opn, owt,
              ibuf, b0, b1, b2, b3, sem):
    cid = lax.axis_index("c")
    sid = lax.axis_index("s")
    wid = sid * 2 + cid
    start = pl.multiple_of(wid * jnp.int32(_SCHUNK), 8)
    sl = pl.ds(start, _SCHUNK)
    pltpu.sync_copy(idx.at[sl], ibuf)
    pltpu.async_copy(nin.at[ibuf], b0, sem).wait()
    pltpu.async_copy(nout.at[ibuf], b1, sem).wait()
    pltpu.async_copy(pn.at[ibuf], b2, sem).wait()
    pltpu.async_copy(wts.at[ibuf], b3, sem).wait()
    pltpu.sync_copy(b0, oin.at[sl])
    pltpu.sync_copy(b1, oout.at[sl])
    pltpu.sync_copy(b2, opn.at[sl])
    pltpu.sync_copy(b3, owt.at[sl])


def _sc_select(node_in, node_out, prob_n, weights, index):
    mesh = plsc.VectorSubcoreMesh(core_axis_name="c", subcore_axis_name="s")
    f = pl.kernel(
        _sel_body,
        out_type=(
            jax.ShapeDtypeStruct((NUM_SAMPLE,), jnp.int32),
            jax.ShapeDtypeStruct((NUM_SAMPLE,), jnp.int32),
            jax.ShapeDtypeStruct((NUM_SAMPLE,), jnp.float32),
            jax.ShapeDtypeStruct((NUM_SAMPLE,), jnp.float32),
        ),
        mesh=mesh,
        compiler_params=pltpu.CompilerParams(needs_layout_passes=False),
        scratch_types=[
            pltpu.VMEM((_SCHUNK,), jnp.int32),
            pltpu.VMEM((_SCHUNK,), jnp.int32),
            pltpu.VMEM((_SCHUNK,), jnp.int32),
            pltpu.VMEM((_SCHUNK,), jnp.float32),
            pltpu.VMEM((_SCHUNK,), jnp.float32),
            pltpu.SemaphoreType.DMA,
        ],
    )
    return f(node_in, node_out, prob_n, weights, index)


def _one_pass(shift, key0, pin, pout, tid,
              kbuf, pbuf, kob, pob, posb, hist2, cur, hmatbuf, scr, hmat,
              spay, sem):
    lane = lax.iota(jnp.int32, 16)
    base = tid * jnp.int32(_CHUNK)
    shift = jnp.int32(shift)

    zero16 = jnp.zeros((16,), jnp.int32)
    ones16 = jnp.ones((16,), jnp.int32)

    def zero_vreg(j, off):
        hist2[pl.ds(off, 16)] = zero16
        return off + 16

    lax.fori_loop(0, _NB, zero_vreg, jnp.int32(0))

    def hist_vreg(j, off):
        k = kbuf[pl.ds(off, 16)]
        dig = lax.shift_right_logical(k, shift) & 255
        plsc.addupdate_scatter(hist2, [lane * _NB + dig], ones16)
        return off + 16

    def hist_win(w, off):
        pltpu.sync_copy(pin.at[pl.ds(pl.multiple_of(base + off, 8), _W)], pbuf)
        pltpu.async_copy(key0.at[pbuf], kbuf, sem).wait()
        lax.fori_loop(0, _NVREG, hist_vreg, jnp.int32(0))
        return off + _W

    lax.fori_loop(0, _NWIN, hist_win, jnp.int32(0))

    def red_lane(l, c):
        loff, g16, tot = c
        return (loff + _NB, g16, tot + hist2[pl.ds(loff + g16, 16)])

    def red_grp(g, g16):
        _, _, tot = lax.fori_loop(0, 16, red_lane, (jnp.int32(0), g16, zero16))
        kob[pl.ds(g16, 16)] = tot
        return g16 + 16

    lax.fori_loop(0, _NB // 16, red_grp, jnp.int32(0))

    pltpu.sync_copy(kob.at[pl.ds(0, _NB)], hmat.at[tid])
    plsc.subcore_barrier()
    pltpu.sync_copy(hmat, hmatbuf)
    plsc.subcore_barrier()

    def scan_tile(t, c):
        ti, g16, tot, pre = c
        row = hmatbuf[ti, pl.ds(g16, 16)]
        return (ti + 1, g16, tot + row,
                pre + jnp.where(ti < tid, row, zero16))

    def scan_grp(g, c):
        g16, carry = c
        _, _, tot, pre = lax.fori_loop(
            0, _NT, scan_tile, (jnp.int32(0), g16, zero16, zero16))
        cs = plsc.cumsum(tot)
        excl = (cs - tot) + carry
        cur[pl.ds(g16, 16)] = excl + pre
        return (g16 + 16, carry + lax.reduce_max(cs, (0,)))

    lax.fori_loop(0, _NB // 16, scan_grp, (jnp.int32(0), jnp.int32(0)))

    lane_m1 = jnp.maximum(lane - 1, 0)

    def rank_vreg(j, off):
        k = kbuf[pl.ds(off, 16)]
        p = pbuf[pl.ds(off, 16)]
        dig = lax.shift_right_logical(k, shift) & 255
        ck = dig * 16 + lane
        ck_s, p_s = plsc.sort_key_val(ck, p)
        dig_s = lax.shift_right_logical(ck_s, jnp.int32(4))
        scr[...] = dig_s
        prev = plsc.load_gather(scr, [lane_m1])
        head = (lane == 0) | (dig_s != prev)
        start = plsc.cummax(jnp.where(head, lane, 0))
        rank = lane - start
        c = plsc.load_gather(cur, [dig_s])
        pos = c + rank
        pob[pl.ds(off, 16)] = p_s
        posb[pl.ds(off, 16)] = pos
        plsc.addupdate_scatter(cur, [dig_s], ones16)
        return off + 16

    def rank_win(w, off):
        pltpu.sync_copy(pin.at[pl.ds(pl.multiple_of(base + off, 8), _W)], pbuf)
        pltpu.async_copy(key0.at[pbuf], kbuf, sem).wait()
        lax.fori_loop(0, _NVREG, rank_vreg, jnp.int32(0))
        pltpu.sync_copy(pob, spay.at[posb])
        return off + _W

    lax.fori_loop(0, _NWIN, rank_win, jnp.int32(0))

    plsc.subcore_barrier()

    def back_win(w, off):
        start = pl.multiple_of(base + off, 8)
        pltpu.sync_copy(spay.at[pl.ds(start, _W)], pbuf)
        pltpu.sync_copy(pbuf, pout.at[pl.ds(start, _W)])
        return off + _W

    lax.fori_loop(0, _NWIN, back_win, jnp.int32(0))

    plsc.subcore_barrier()


def _sort_body(kin, pin, pout_hbm,
               kbuf, pbuf, kob, pob, posb, hist2, cur, hmatbuf, scr, hmat,
               spay, sem):
    cid = lax.axis_index("c")
    tid = lax.axis_index("s")

    @pl.when(cid == 0)
    def _():
        args = (tid, kbuf, pbuf, kob, pob, posb, hist2, cur, hmatbuf, scr,
                hmat, spay, sem)
        _one_pass(0, kin, pin, pout_hbm, *args)
        _one_pass(8, kin, pout_hbm, pout_hbm, *args)
        _one_pass(16, kin, pout_hbm, pout_hbm, *args)
        _one_pass(24, kin, pout_hbm, pout_hbm, *args)


def _sc_radix_sort(keys, payload):
    mesh = plsc.VectorSubcoreMesh(core_axis_name="c", subcore_axis_name="s")
    f = pl.kernel(
        _sort_body,
        out_type=jax.ShapeDtypeStruct((NUM_EDGE,), jnp.int32),
        mesh=mesh,
        compiler_params=pltpu.CompilerParams(needs_layout_passes=False),
        scratch_types=[
            pltpu.VMEM((_W,), jnp.int32),
            pltpu.VMEM((_W,), jnp.int32),
            pltpu.VMEM((_NB,), jnp.int32),
            pltpu.VMEM((_W,), jnp.int32),
            pltpu.VMEM((_W,), jnp.int32),
            pltpu.VMEM((_NB * 16,), jnp.int32),
            pltpu.VMEM((_NB,), jnp.int32),
            pltpu.VMEM((_NT, _NB), jnp.int32),
            pltpu.VMEM((16,), jnp.int32),
            pltpu.VMEM_SHARED((_NT, _NB), jnp.int32),
            pltpu.VMEM_SHARED((NUM_EDGE,), jnp.int32),
            pltpu.SemaphoreType.DMA,
        ],
    )
    return f(keys, payload)


def kernel(edge_index, edge_weight):
    node_in = edge_index[0].astype(jnp.int32)
    node_out = edge_index[1].astype(jnp.int32)

    degree_in, degree_out = _sc_degrees(node_in, node_out, edge_weight)

    prob = _sc_prob(1.0 / degree_in, 1.0 / degree_out, node_in, node_out)
    m = jnp.mean(prob)

    u = jax.random.uniform(jax.random.key(42), (NUM_EDGE,), dtype=jnp.float32,
                           minval=1e-20, maxval=1.0)
    gumbel = -jnp.log(-jnp.log(u))

    s, prob_n, key = _scores(prob, m, gumbel)

    perm = _sc_radix_sort(key, jnp.arange(NUM_EDGE, dtype=jnp.int32))
    index = perm[:NUM_SAMPLE]

    new_in, new_out, sel_p, sel_w = _sc_select(
        node_in, node_out, prob_n, edge_weight, index)
    new_edge_index = jnp.stack([new_in, new_out]).astype(jnp.int64)
    new_edge_weight = sel_w / (NUM_SAMPLE * sel_p / NUM_EDGE)
    return new_edge_index, new_edge_weight
